# Initial kernel scaffold; baseline (speedup 1.0000x reference)
import jax
import jax.numpy as jnp
from jax.experimental import pallas as pl


def kernel(node_feature, edge_feature, vdw_radii, distance, interaction_type, edge_index, n_rot, params):
    raise NotImplementedError("write your pallas kernel here")


# speedup vs baseline: 10.7661x; 10.7661x over previous
import functools

import jax
import jax.numpy as jnp
from jax import lax
from jax.experimental import pallas as pl
from jax.experimental.pallas import tpu as pltpu
from jax.experimental.pallas import tpu_sc as plsc

N = 10000
E = 160000
HID = 128
HEADS = 8
DH = HID // HEADS
NLAYERS = 10
FFN = 2 * HID
CUT = 6.5
f32 = jnp.float32

NC = 2
NS = 16
NW = NC * NS
PER_W = E // NW
CH = 128
NFULL = PER_W // CH
TAIL = PER_W - NFULL * CH
NPAD = 10240
TROWS = NPAD // NS
ZR = 128

EB = 1000
NB = 1000

_PREC = jax.lax.Precision.HIGHEST


def _dot(a, b):
    return jnp.dot(a, b, preferred_element_type=f32, precision=_PREC)


def _lnk(x, g, b):
    m = jnp.mean(x, axis=-1, keepdims=True)
    v = jnp.mean((x - m) * (x - m), axis=-1, keepdims=True)
    return (x - m) * lax.rsqrt(v + 1e-5) * g + b


def _sel_hd():
    r = lax.broadcasted_iota(jnp.int32, (HID, HEADS), 0) // DH
    c = lax.broadcasted_iota(jnp.int32, (HID, HEADS), 1)
    return (r == c).astype(f32)


def _sel_dh():
    r = lax.broadcasted_iota(jnp.int32, (HEADS, HID), 0)
    c = lax.broadcasted_iota(jnp.int32, (HEADS, HID), 1) // DH
    return (r == c).astype(f32)


def _rowspec(b, w):
    return pl.BlockSpec((b, w), lambda i: (i, 0))


def _fullspec(shape):
    return pl.BlockSpec(shape, lambda i: tuple(0 for _ in shape))


@functools.lru_cache(maxsize=None)
def _make_gather(wa, wb):
    mesh = plsc.VectorSubcoreMesh(core_axis_name="c", subcore_axis_name="s")

    @functools.partial(
        pl.kernel, mesh=mesh,
        out_type=(jax.ShapeDtypeStruct((E, wa), f32),
                  jax.ShapeDtypeStruct((E, wb), f32)),
        scratch_types=(
            pltpu.VMEM((CH,), jnp.int32), pltpu.VMEM((CH,), jnp.int32),
            pltpu.VMEM((TAIL,), jnp.int32), pltpu.VMEM((TAIL,), jnp.int32),
            pltpu.VMEM((CH, wa), f32), pltpu.VMEM((CH, wb), f32),
            pltpu.VMEM((TAIL, wa), f32), pltpu.VMEM((TAIL, wb), f32),
            pltpu.SemaphoreType.DMA, pltpu.SemaphoreType.DMA,
        ),
    )
    def gk(tabA, tabB, idxA, idxB, outA, outB,
           ia, ib, ta, tb, ra, rb, tra, trb, sa, sb):
        wid = lax.axis_index("s") * NC + lax.axis_index("c")
        base = wid * PER_W

        def body(i, carry):
            off = base + i * CH
            pltpu.sync_copy(idxA.at[pl.ds(off, CH)], ia)
            pltpu.sync_copy(idxB.at[pl.ds(off, CH)], ib)
            ca = pltpu.async_copy(tabA.at[ia], ra, sa)
            cb = pltpu.async_copy(tabB.at[ib], rb, sb)
            ca.wait()
            cb.wait()
            pltpu.sync_copy(ra, outA.at[pl.ds(off, CH)])
            pltpu.sync_copy(rb, outB.at[pl.ds(off, CH)])
            return carry

        lax.fori_loop(0, NFULL, body, 0)
        off = base + NFULL * CH
        pltpu.sync_copy(idxA.at[pl.ds(off, TAIL)], ta)
        pltpu.sync_copy(idxB.at[pl.ds(off, TAIL)], tb)
        ca = pltpu.async_copy(tabA.at[ta], tra, sa)
        cb = pltpu.async_copy(tabB.at[tb], trb, sb)
        ca.wait()
        cb.wait()
        pltpu.sync_copy(tra, outA.at[pl.ds(off, TAIL)])
        pltpu.sync_copy(trb, outB.at[pl.ds(off, TAIL)])

    return gk


def _gather_layer(kv_t, q_t, src, dst):
    return _make_gather(2 * HID, HID)(kv_t, q_t, src, dst)


def _gather_fin(vfin_a, vfin_b, src, dst):
    return _make_gather(2 * HID, 2 * HID)(vfin_a, vfin_b, src, dst)


@functools.lru_cache(maxsize=None)
def _make_scatter():
    mesh = plsc.VectorSubcoreMesh(core_axis_name="c", subcore_axis_name="s")

    @functools.partial(
        pl.kernel, mesh=mesh,
        out_type=(jax.ShapeDtypeStruct((2 * NPAD, HID), f32),
                  jax.ShapeDtypeStruct((2 * NPAD, HID), f32)),
        scratch_types=(
            pltpu.VMEM((CH,), jnp.int32),
            pltpu.VMEM((CH, HID), f32),
            pltpu.VMEM((TAIL,), jnp.int32),
            pltpu.VMEM((TAIL, HID), f32),
            pltpu.VMEM((ZR, HID), f32),
            pltpu.VMEM_SHARED((NPAD, HID), f32),
        ),
    )
    def sck(contrib_h, arep_h, dst_h, agg_out, den_out,
            idx_v, rows_v, idx_t, rows_t, zb, acc_sh):
        c = lax.axis_index("c")
        s = lax.axis_index("s")
        zeros16 = jnp.zeros((16,), f32)

        def zrow(r, carry):
            def zcol(j, cc):
                zb[r, pl.ds(j * 16, 16)] = zeros16
                return cc
            return lax.fori_loop(0, HID // 16, zcol, carry)

        lax.fori_loop(0, ZR, zrow, 0)

        r0 = s * TROWS
        wid = s * NC + c
        base = wid * PER_W

        def zero_acc():
            for b in range(TROWS // ZR):
                pltpu.sync_copy(zb, acc_sh.at[pl.ds(r0 + b * ZR, ZR)])

        def scatter_pass(src_h, out_h):
            def body(i, carry):
                off = base + i * CH
                pltpu.sync_copy(dst_h.at[pl.ds(off, CH)], idx_v)
                pltpu.sync_copy(src_h.at[pl.ds(off, CH)], rows_v)
                pltpu.sync_copy(rows_v, acc_sh.at[idx_v], add=True)
                return carry

            lax.fori_loop(0, NFULL, body, 0)
            off = base + NFULL * CH
            pltpu.sync_copy(dst_h.at[pl.ds(off, TAIL)], idx_t)
            pltpu.sync_copy(src_h.at[pl.ds(off, TAIL)], rows_t)
            pltpu.sync_copy(rows_t, acc_sh.at[idx_t], add=True)
            plsc.subcore_barrier()
            for b in range(TROWS // CH):
                pltpu.sync_copy(acc_sh.at[pl.ds(r0 + b * CH, CH)], rows_v)
                pltpu.sync_copy(rows_v,
                                out_h.at[pl.ds(c * NPAD + r0 + b * CH, CH)])
            plsc.subcore_barrier()

        zero_acc()
        plsc.subcore_barrier()
        scatter_pass(contrib_h, agg_out)
        zero_acc()
        plsc.subcore_barrier()
        scatter_pass(arep_h, den_out)

    return sck


def _scatter(contrib, arep, dst):
    return _make_scatter()(contrib, arep, dst)


def _node_embed_body(nf_r, Wv_r, bv_r, Wq_r, bq_r, Wk_r, bk_r, Wvv_r, bvv_r,
                     v_r, q_r, kv_r):
    v = _dot(nf_r[...], Wv_r[...]) + bv_r[...]
    v_r[...] = v
    q_r[...] = _dot(v, Wq_r[...]) + bq_r[...]
    k = _dot(v, Wk_r[...]) + bk_r[...]
    val = _dot(v, Wvv_r[...]) + bvv_r[...]
    kv_r[...] = jnp.concatenate([k, val], axis=1)


def _node_embed(nf_pad, Wv, bv, Wq, bq, Wk, bk, Wvv, bvv):
    return pl.pallas_call(
        _node_embed_body, grid=(N // NB,),
        in_specs=[_rowspec(NB, 40), _fullspec((40, HID)), _fullspec((1, HID)),
                  _fullspec((HID, HID)), _fullspec((1, HID)),
                  _fullspec((HID, HID)), _fullspec((1, HID)),
                  _fullspec((HID, HID)), _fullspec((1, HID))],
        out_specs=[_rowspec(NB, HID), _rowspec(NB, HID), _rowspec(NB, 2 * HID)],
        out_shape=[jax.ShapeDtypeStruct((N, HID), f32),
                   jax.ShapeDtypeStruct((N, HID), f32),
                   jax.ShapeDtypeStruct((N, 2 * HID), f32)],
    )(nf_pad, Wv, bv, Wq, bq, Wk, bk, Wvv, bvv)


def _edge_embed_body(ef_r, d_r, We_r, be_r, e_r, env_r):
    e_r[...] = _dot(ef_r[...], We_r[...]) + be_r[...]
    d = d_r[...]
    x01 = jnp.clip(d / CUT, 0.0, 1.0)
    x2 = x01 * x01
    x3 = x2 * x01
    x4 = x2 * x2
    x5 = x4 * x01
    env_r[...] = jnp.where(d < CUT, 1.0 - 6.0 * x5 + 15.0 * x4 - 10.0 * x3, 0.0)


def _edge_embed(ef_pad, dist, We, be):
    return pl.pallas_call(
        _edge_embed_body, grid=(E // EB,),
        in_specs=[_rowspec(EB, 40), _rowspec(EB, 1),
                  _fullspec((40, HID)), _fullspec((1, HID))],
        out_specs=[_rowspec(EB, HID), _rowspec(EB, 1)],
        out_shape=[jax.ShapeDtypeStruct((E, HID), f32),
                   jax.ShapeDtypeStruct((E, 1), f32)],
    )(ef_pad, dist, We, be)


def _edge_layer_body(e_r, ksvs_r, qd_r, env_r,
                     Wee_r, bee_r, Woe_r, boe_r, g1_r, b1_r,
                     Wf1_r, bf1_r, Wf2_r, bf2_r, g2_r, b2_r,
                     e2_r, contrib_r, arep_r):
    e = e_r[...]
    ks = ksvs_r[:, :HID]
    vs = ksvs_r[:, HID:]
    qd = qd_r[...]
    ee = _dot(e, Wee_r[...]) + bee_r[...]
    score = ks * qd * (0.25 * ee)
    s = jnp.clip(_dot(score, _sel_hd()), -5.0, 5.0)
    a = jnp.exp(s) * env_r[...]
    arep = _dot(a, _sel_dh())
    contrib_r[...] = arep * vs
    arep_r[...] = arep
    eh = e + _dot(score, Woe_r[...]) + boe_r[...]
    eh = _lnk(eh, g1_r[...], b1_r[...])
    h = jnp.maximum(_dot(eh, Wf1_r[...]) + bf1_r[...], 0.0)
    e2_r[...] = _lnk(eh + _dot(h, Wf2_r[...]) + bf2_r[...], g2_r[...], b2_r[...])


def _edge_layer(e, ksvs, qd, env, Wee, bee, Woe, boe, g1, b1,
                Wf1, bf1, Wf2, bf2, g2, b2):
    return pl.pallas_call(
        _edge_layer_body, grid=(E // EB,),
        in_specs=[_rowspec(EB, HID), _rowspec(EB, 2 * HID), _rowspec(EB, HID),
                  _rowspec(EB, 1),
                  _fullspec((HID, HID)), _fullspec((1, HID)),
                  _fullspec((HID, HID)), _fullspec((1, HID)),
                  _fullspec((1, HID)), _fullspec((1, HID)),
                  _fullspec((HID, FFN)), _fullspec((1, FFN)),
                  _fullspec((FFN, HID)), _fullspec((1, HID)),
                  _fullspec((1, HID)), _fullspec((1, HID))],
        out_specs=[_rowspec(EB, HID), _rowspec(EB, HID), _rowspec(EB, HID)],
        out_shape=[jax.ShapeDtypeStruct((E, HID), f32),
                   jax.ShapeDtypeStruct((E, HID), f32),
                   jax.ShapeDtypeStruct((E, HID), f32)],
    )(e, ksvs, qd, env, Wee, bee, Woe, boe, g1, b1, Wf1, bf1, Wf2, bf2, g2, b2)


def _node_common(v_r, aggp_r, denp_r, Wo_r, bo_r, g1_r, b1_r,
                 Wf1_r, bf1_r, Wf2_r, bf2_r, g2_r, b2_r):
    agg = aggp_r[0] + aggp_r[1]
    denr = denp_r[0] + denp_r[1] + 1e-6
    v_att = agg / denr
    vh = v_r[...] + _dot(v_att, Wo_r[...]) + bo_r[...]
    vh = _lnk(vh, g1_r[...], b1_r[...])
    h = jnp.maximum(_dot(vh, Wf1_r[...]) + bf1_r[...], 0.0)
    return _lnk(vh + _dot(h, Wf2_r[...]) + bf2_r[...], g2_r[...], b2_r[...])


def _node_layer_body(v_r, aggp_r, denp_r, Wo_r, bo_r, g1_r, b1_r,
                     Wf1_r, bf1_r, Wf2_r, bf2_r, g2_r, b2_r,
                     Wq_r, bq_r, Wk_r, bk_r, Wvv_r, bvv_r,
                     v2_r, q_r, kv_r):
    v2 = _node_common(v_r, aggp_r, denp_r, Wo_r, bo_r, g1_r, b1_r,
                      Wf1_r, bf1_r, Wf2_r, bf2_r, g2_r, b2_r)
    v2_r[...] = v2
    q_r[...] = _dot(v2, Wq_r[...]) + bq_r[...]
    k = _dot(v2, Wk_r[...]) + bk_r[...]
    val = _dot(v2, Wvv_r[...]) + bvv_r[...]
    kv_r[...] = jnp.concatenate([k, val], axis=1)


def _node_layer(v, aggp, denp, Wo, bo, g1, b1, Wf1, bf1, Wf2, bf2, g2, b2,
                Wq, bq, Wk, bk, Wvv, bvv):
    wspecs = [_fullspec((HID, HID)), _fullspec((1, HID)),
              _fullspec((1, HID)), _fullspec((1, HID)),
              _fullspec((HID, FFN)), _fullspec((1, FFN)),
              _fullspec((FFN, HID)), _fullspec((1, HID)),
              _fullspec((1, HID)), _fullspec((1, HID))]
    qkvspecs = [_fullspec((HID, HID)), _fullspec((1, HID)),
                _fullspec((HID, HID)), _fullspec((1, HID)),
                _fullspec((HID, HID)), _fullspec((1, HID))]
    return pl.pallas_call(
        _node_layer_body, grid=(N // NB,),
        in_specs=[_rowspec(NB, HID),
                  pl.BlockSpec((2, NB, HID), lambda i: (0, i, 0)),
                  pl.BlockSpec((2, NB, HID), lambda i: (0, i, 0))]
                 + wspecs + qkvspecs,
        out_specs=[_rowspec(NB, HID), _rowspec(NB, HID), _rowspec(NB, 2 * HID)],
        out_shape=[jax.ShapeDtypeStruct((N, HID), f32),
                   jax.ShapeDtypeStruct((N, HID), f32),
                   jax.ShapeDtypeStruct((N, 2 * HID), f32)],
    )(v, aggp, denp, Wo, bo, g1, b1, Wf1, bf1, Wf2, bf2, g2, b2,
      Wq, bq, Wk, bk, Wvv, bvv)


def _node_final_body(v_r, aggp_r, denp_r, nf0_r, rad_r,
                     Wo_r, bo_r, g1_r, b1_r, Wf1_r, bf1_r, Wf2_r, bf2_r,
                     g2_r, b2_r, vfin_r, z_r):
    v2 = _node_common(v_r, aggp_r, denp_r, Wo_r, bo_r, g1_r, b1_r,
                      Wf1_r, bf1_r, Wf2_r, bf2_r, g2_r, b2_r)
    vsc = v2 * nf0_r[...]
    vfin_r[:, :HID] = vsc
    lane = lax.broadcasted_iota(jnp.int32, (1, HID), 1)
    vfin_r[:, HID:] = jnp.where(lane == 0, rad_r[...], 0.0)

    @pl.when(pl.program_id(0) == 0)
    def _():
        z_r[...] = jnp.zeros_like(z_r)

    z_r[...] += jnp.sum(vsc, axis=0, keepdims=True)


def _node_final(v, aggp, denp, nf0, rad, Wo, bo, g1, b1, Wf1, bf1, Wf2, bf2,
                g2, b2):
    wspecs = [_fullspec((HID, HID)), _fullspec((1, HID)),
              _fullspec((1, HID)), _fullspec((1, HID)),
              _fullspec((HID, FFN)), _fullspec((1, FFN)),
              _fullspec((FFN, HID)), _fullspec((1, HID)),
              _fullspec((1, HID)), _fullspec((1, HID))]
    return pl.pallas_call(
        _node_final_body, grid=(N // NB,),
        in_specs=[_rowspec(NB, HID),
                  pl.BlockSpec((2, NB, HID), lambda i: (0, i, 0)),
                  pl.BlockSpec((2, NB, HID), lambda i: (0, i, 0)),
                  _rowspec(NB, 1), _rowspec(NB, 1)] + wspecs,
        out_specs=[_rowspec(NB, 2 * HID), _fullspec((1, HID))],
        out_shape=[jax.ShapeDtypeStruct((N, 2 * HID), f32),
                   jax.ShapeDtypeStruct((1, HID), f32)],
    )(v, aggp, denp, nf0, rad, Wo, bo, g1, b1, Wf1, bf1, Wf2, bf2, g2, b2)


def _sigmoid(x):
    return 1.0 / (1.0 + jnp.exp(-x))


def _readout_body(xs_r, xd_r, d_r, it_r, ef0_r,
                  W21_r, b21_r, W22_r, b22_r, W31_r, b31_r, W32_r, b32_r,
                  y_r):
    x = xs_r[:, :HID] + xd_r[:, :HID]
    rsum = xs_r[:, HID:HID + 1] + xd_r[:, HID:HID + 1]
    dist = d_r[...]
    d = dist - rsum
    t = d * 1.25
    V0 = -0.045 * jnp.exp(-(t * t))
    V1 = 0.8 * jnp.where(d < 0, d * d, 0.0)
    V2 = -0.035 * (jnp.where((d > 0) & (d < 2.5), -0.4 * (d - 2.5), 0.0)
                   + jnp.where(d <= 0, 1.0, 0.0))
    V3 = -0.6 * (jnp.where((d > -0.6) & (d < 0), (-5.0 / 3.0) * d, 0.0)
                 + jnp.where(d <= -0.6, 1.0, 0.0))
    mask = jnp.where(dist < CUT, 1.0, 0.0) * ef0_r[...]
    w2 = _sigmoid(_dot(jnp.maximum(_dot(x, W21_r[...]) + b21_r[...], 0.0),
                       W22_r[...]) + b22_r[...]) + 0.5
    w3 = _sigmoid(_dot(jnp.maximum(_dot(x, W31_r[...]) + b31_r[...], 0.0),
                       W32_r[...]) + b32_r[...]) + 0.5
    it1 = it_r[:, 1:2]
    it2 = it_r[:, 2:3]
    t2 = (w2[:, 0:1] * V0 + w2[:, 1:2] * V1 + it1 * w2[:, 2:3] * V2
          + it2 * w2[:, 3:4] * V3)
    t3 = (w3[:, 0:1] * V0 + w3[:, 1:2] * V1 + it1 * w3[:, 2:3] * V2
          + it2 * w3[:, 3:4] * V3)
    p2 = jnp.sum(mask * t2) * 0.5
    p3 = jnp.sum(mask * t3) * 0.5

    @pl.when(pl.program_id(0) == 0)
    def _():
        y_r[...] = jnp.zeros_like(y_r)

    lane = lax.broadcasted_iota(jnp.int32, (1, HID), 1)
    y_r[...] += jnp.where(lane == 0, p2, 0.0) + jnp.where(lane == 1, p3, 0.0)


def _readout(xs, xd, dist, itp, ef0, W21, b21, W22, b22, W31, b31, W32, b32):
    return pl.pallas_call(
        _readout_body, grid=(E // EB,),
        in_specs=[_rowspec(EB, 2 * HID), _rowspec(EB, 2 * HID), _rowspec(EB, 1),
                  _rowspec(EB, 8), _rowspec(EB, 1),
                  _fullspec((HID, HID)), _fullspec((1, HID)),
                  _fullspec((HID, 8)), _fullspec((1, 8)),
                  _fullspec((HID, HID)), _fullspec((1, HID)),
                  _fullspec((HID, 8)), _fullspec((1, 8))],
        out_specs=_fullspec((1, HID)),
        out_shape=jax.ShapeDtypeStruct((1, HID), f32),
    )(xs, xd, dist, itp, ef0, W21, b21, W22, b22, W31, b31, W32, b32)


def _head_body(z_r, y_r, nrot_r, W11_r, b11_r, W12_r, b12_r,
               W41_r, b41_r, W42_r, b42_r, out_r):
    z = z_r[...]
    y1 = (_dot(jnp.maximum(_dot(z, W11_r[...]) + b11_r[...], 0.0),
               W12_r[...]) + b12_r[...])[0, 0]
    w4 = _sigmoid((_dot(jnp.maximum(_dot(z, W41_r[...]) + b41_r[...], 0.0),
                        W42_r[...]) + b42_r[...])[0, 0]) + 0.5
    y2 = y_r[0, 0]
    y3 = y_r[0, 1] / (1.0 + w4 * 0.05846 * nrot_r[0, 0])
    lane = lax.broadcasted_iota(jnp.int32, (1, HID), 1)
    out_r[...] = (jnp.where(lane == 0, y1, 0.0)
                  + jnp.where(lane == 1, y2, 0.0)
                  + jnp.where(lane == 2, y3, 0.0))


def _head(z, y, nrot, W11, b11, W12, b12, W41, b41, W42, b42):
    return pl.pallas_call(
        _head_body,
        out_shape=jax.ShapeDtypeStruct((1, HID), f32),
    )(z, y, nrot, W11, b11, W12, b12, W41, b41, W42, b42)


def kernel(node_feature, edge_feature, vdw_radii, distance, interaction_type,
           edge_index, n_rot, params):
    p = params
    src = edge_index[0].astype(jnp.int32)
    dst = edge_index[1].astype(jnp.int32)
    nf_pad = jnp.pad(node_feature, ((0, 0), (0, 1)))
    ef_pad = jnp.pad(edge_feature, ((0, 0), (0, 1)))
    Wv = jnp.pad(p['Wv_emb'], ((0, 1), (0, 0)))
    We = jnp.pad(p['We_emb'], ((0, 1), (0, 0)))
    dist = distance[:, None]
    itp = jnp.pad(interaction_type, ((0, 0), (0, 5)))
    ef0 = edge_feature[:, 0:1]
    nf0 = node_feature[:, 0:1]
    rad = vdw_radii[:, None]
    nrot = jnp.asarray(n_rot, f32).reshape(1, 1)

    def row(x):
        return x.reshape(1, -1)

    v, q_t, kv_t = _node_embed(nf_pad, Wv, row(p['bv_emb']),
                               p['Wq'][0], row(p['bq'][0]),
                               p['Wk'][0], row(p['bk'][0]),
                               p['Wvv'][0], row(p['bvv'][0]))
    e, env = _edge_embed(ef_pad, dist, We, row(p['be_emb']))

    vfin = None
    z = None
    for l in range(NLAYERS):
        ksvs, qd = _gather_layer(kv_t, q_t, src, dst)
        e, contrib, arep = _edge_layer(
            e, ksvs, qd, env,
            p['Wee'][l], row(p['bee'][l]), p['Woe'][l], row(p['boe'][l]),
            row(p['g1e'][l]), row(p['b1e'][l]),
            p['Wf1e'][l], row(p['bf1e'][l]), p['Wf2e'][l], row(p['bf2e'][l]),
            row(p['g2e'][l]), row(p['b2e'][l]))
        aggf, denf = _scatter(contrib, arep, dst)
        aggp = aggf.reshape(2, NPAD, HID)
        denp = denf.reshape(2, NPAD, HID)
        nw = (p['Wo'][l], row(p['bo'][l]), row(p['g1v'][l]), row(p['b1v'][l]),
              p['Wf1v'][l], row(p['bf1v'][l]), p['Wf2v'][l], row(p['bf2v'][l]),
              row(p['g2v'][l]), row(p['b2v'][l]))
        if l < NLAYERS - 1:
            v, q_t, kv_t = _node_layer(v, aggp, denp, *nw,
                                       p['Wq'][l + 1], row(p['bq'][l + 1]),
                                       p['Wk'][l + 1], row(p['bk'][l + 1]),
                                       p['Wvv'][l + 1], row(p['bvv'][l + 1]))
        else:
            vfin, z = _node_final(v, aggp, denp, nf0, rad, *nw)

    xs, xd = _gather_fin(vfin, vfin, src, dst)
    y23 = _readout(xs, xd, dist, itp, ef0,
                   p['r2_W1'], row(p['r2_b1']),
                   jnp.pad(p['r2_W2'], ((0, 0), (0, 4))),
                   row(jnp.pad(p['r2_b2'], (0, 4))),
                   p['r3_W1'], row(p['r3_b1']),
                   jnp.pad(p['r3_W2'], ((0, 0), (0, 4))),
                   row(jnp.pad(p['r3_b2'], (0, 4))))
    out = _head(z, y23, nrot,
                p['r1_W1'], row(p['r1_b1']),
                jnp.pad(p['r1_W2'], ((0, 0), (0, 7))),
                row(jnp.pad(p['r1_b2'], (0, 7))),
                p['r4_W1'], row(p['r4_b1']),
                jnp.pad(p['r4_W2'], ((0, 0), (0, 7))),
                row(jnp.pad(p['r4_b2'], (0, 7))))
    return out[0, :3]


# speedup vs baseline: 27.2326x; 2.5295x over previous
import functools

import jax
import jax.numpy as jnp
from jax import lax
from jax.experimental import pallas as pl
from jax.experimental.pallas import tpu as pltpu
from jax.experimental.pallas import tpu_sc as plsc

N = 10000
E = 160000
HID = 128
HEADS = 8
DH = HID // HEADS
NLAYERS = 10
FFN = 2 * HID
CUT = 6.5
f32 = jnp.float32

NC = 2
NS = 16
NW = NC * NS
PER_W = E // NW
CH = 128
NFULL = PER_W // CH
TAIL = PER_W - NFULL * CH
NPAD = 10240
TROWS = NPAD // NS
ZR = 128

EB = 2000
NB = 1000

_PREC = jax.lax.Precision.DEFAULT


def _dot(a, b):
    return jnp.dot(a, b, preferred_element_type=f32, precision=_PREC)


def _lnk(x, g, b):
    m = jnp.mean(x, axis=-1, keepdims=True)
    v = jnp.mean((x - m) * (x - m), axis=-1, keepdims=True)
    return (x - m) * lax.rsqrt(v + 1e-5) * g + b


def _sel_hd():
    r = lax.broadcasted_iota(jnp.int32, (HID, HEADS), 0) // DH
    c = lax.broadcasted_iota(jnp.int32, (HID, HEADS), 1)
    return (r == c).astype(f32)


def _sel_dh():
    r = lax.broadcasted_iota(jnp.int32, (HEADS, HID), 0)
    c = lax.broadcasted_iota(jnp.int32, (HEADS, HID), 1) // DH
    return (r == c).astype(f32)


def _rowspec(b, w):
    return pl.BlockSpec((b, w), lambda i: (i, 0))


def _fullspec(shape):
    return pl.BlockSpec(shape, lambda i: tuple(0 for _ in shape))


@functools.lru_cache(maxsize=None)
def _make_gather(wa, wb):
    mesh = plsc.VectorSubcoreMesh(core_axis_name="c", subcore_axis_name="s")

    @functools.partial(
        pl.kernel, mesh=mesh,
        out_type=(jax.ShapeDtypeStruct((E, wa), f32),
                  jax.ShapeDtypeStruct((E, wb), f32)),
        scratch_types=(
            pltpu.VMEM((CH,), jnp.int32), pltpu.VMEM((CH,), jnp.int32),
            pltpu.VMEM((TAIL,), jnp.int32), pltpu.VMEM((TAIL,), jnp.int32),
            pltpu.VMEM((CH, wa), f32), pltpu.VMEM((CH, wb), f32),
            pltpu.VMEM((TAIL, wa), f32), pltpu.VMEM((TAIL, wb), f32),
            pltpu.SemaphoreType.DMA, pltpu.SemaphoreType.DMA,
        ),
    )
    def gk(tabA, tabB, idxA, idxB, outA, outB,
           ia, ib, ta, tb, ra, rb, tra, trb, sa, sb):
        wid = lax.axis_index("s") * NC + lax.axis_index("c")
        base = wid * PER_W

        def body(i, carry):
            off = base + i * CH
            pltpu.sync_copy(idxA.at[pl.ds(off, CH)], ia)
            pltpu.sync_copy(idxB.at[pl.ds(off, CH)], ib)
            ca = pltpu.async_copy(tabA.at[ia], ra, sa)
            cb = pltpu.async_copy(tabB.at[ib], rb, sb)
            ca.wait()
            cb.wait()
            pltpu.sync_copy(ra, outA.at[pl.ds(off, CH)])
            pltpu.sync_copy(rb, outB.at[pl.ds(off, CH)])
            return carry

        lax.fori_loop(0, NFULL, body, 0)
        off = base + NFULL * CH
        pltpu.sync_copy(idxA.at[pl.ds(off, TAIL)], ta)
        pltpu.sync_copy(idxB.at[pl.ds(off, TAIL)], tb)
        ca = pltpu.async_copy(tabA.at[ta], tra, sa)
        cb = pltpu.async_copy(tabB.at[tb], trb, sb)
        ca.wait()
        cb.wait()
        pltpu.sync_copy(tra, outA.at[pl.ds(off, TAIL)])
        pltpu.sync_copy(trb, outB.at[pl.ds(off, TAIL)])

    return gk


def _gather_layer(kv_t, q_t, src, dst):
    return _make_gather(2 * HID, HID)(kv_t, q_t, src, dst)


def _gather_fin(vfin_a, vfin_b, src, dst):
    return _make_gather(2 * HID, 2 * HID)(vfin_a, vfin_b, src, dst)


@functools.lru_cache(maxsize=None)
def _make_scatter():
    mesh = plsc.VectorSubcoreMesh(core_axis_name="c", subcore_axis_name="s")

    @functools.partial(
        pl.kernel, mesh=mesh,
        out_type=(jax.ShapeDtypeStruct((2 * NPAD, HID), f32),
                  jax.ShapeDtypeStruct((2 * NPAD, HID), f32)),
        scratch_types=(
            pltpu.VMEM((CH,), jnp.int32),
            pltpu.VMEM((CH, HID), f32),
            pltpu.VMEM((TAIL,), jnp.int32),
            pltpu.VMEM((TAIL, HID), f32),
            pltpu.VMEM((ZR, HID), f32),
            pltpu.VMEM_SHARED((NPAD, HID), f32),
        ),
    )
    def sck(contrib_h, arep_h, dst_h, agg_out, den_out,
            idx_v, rows_v, idx_t, rows_t, zb, acc_sh):
        c = lax.axis_index("c")
        s = lax.axis_index("s")
        zeros16 = jnp.zeros((16,), f32)

        def zrow(r, carry):
            def zcol(j, cc):
                zb[r, pl.ds(j * 16, 16)] = zeros16
                return cc
            return lax.fori_loop(0, HID // 16, zcol, carry)

        lax.fori_loop(0, ZR, zrow, 0)

        r0 = s * TROWS
        wid = s * NC + c
        base = wid * PER_W

        def zero_acc():
            for b in range(TROWS // ZR):
                pltpu.sync_copy(zb, acc_sh.at[pl.ds(r0 + b * ZR, ZR)])

        def scatter_pass(src_h, out_h):
            def body(i, carry):
                off = base + i * CH
                pltpu.sync_copy(dst_h.at[pl.ds(off, CH)], idx_v)
                pltpu.sync_copy(src_h.at[pl.ds(off, CH)], rows_v)
                pltpu.sync_copy(rows_v, acc_sh.at[idx_v], add=True)
                return carry

            lax.fori_loop(0, NFULL, body, 0)
            off = base + NFULL * CH
            pltpu.sync_copy(dst_h.at[pl.ds(off, TAIL)], idx_t)
            pltpu.sync_copy(src_h.at[pl.ds(off, TAIL)], rows_t)
            pltpu.sync_copy(rows_t, acc_sh.at[idx_t], add=True)
            plsc.subcore_barrier()
            for b in range(TROWS // CH):
                pltpu.sync_copy(acc_sh.at[pl.ds(r0 + b * CH, CH)], rows_v)
                pltpu.sync_copy(rows_v,
                                out_h.at[pl.ds(c * NPAD + r0 + b * CH, CH)])
            plsc.subcore_barrier()

        zero_acc()
        plsc.subcore_barrier()
        scatter_pass(contrib_h, agg_out)
        zero_acc()
        plsc.subcore_barrier()
        scatter_pass(arep_h, den_out)

    return sck


def _scatter(contrib, arep, dst):
    return _make_scatter()(contrib, arep, dst)


def _node_embed_body(nf_r, Wv_r, bv_r, Wq_r, bq_r, Wk_r, bk_r, Wvv_r, bvv_r,
                     v_r, q_r, kv_r):
    v = _dot(nf_r[...], Wv_r[...]) + bv_r[...]
    v_r[...] = v
    q_r[...] = _dot(v, Wq_r[...]) + bq_r[...]
    k = _dot(v, Wk_r[...]) + bk_r[...]
    val = _dot(v, Wvv_r[...]) + bvv_r[...]
    kv_r[...] = jnp.concatenate([k, val], axis=1)


def _node_embed(nf_pad, Wv, bv, Wq, bq, Wk, bk, Wvv, bvv):
    return pl.pallas_call(
        _node_embed_body, grid=(N // NB,),
        in_specs=[_rowspec(NB, 40), _fullspec((40, HID)), _fullspec((1, HID)),
                  _fullspec((HID, HID)), _fullspec((1, HID)),
                  _fullspec((HID, HID)), _fullspec((1, HID)),
                  _fullspec((HID, HID)), _fullspec((1, HID))],
        out_specs=[_rowspec(NB, HID), _rowspec(NB, HID), _rowspec(NB, 2 * HID)],
        out_shape=[jax.ShapeDtypeStruct((N, HID), f32),
                   jax.ShapeDtypeStruct((N, HID), f32),
                   jax.ShapeDtypeStruct((N, 2 * HID), f32)],
    )(nf_pad, Wv, bv, Wq, bq, Wk, bk, Wvv, bvv)


def _edge_embed_body(ef_r, d_r, We_r, be_r, e_r, env_r):
    e_r[...] = _dot(ef_r[...], We_r[...]) + be_r[...]
    d = d_r[...]
    x01 = jnp.clip(d / CUT, 0.0, 1.0)
    x2 = x01 * x01
    x3 = x2 * x01
    x4 = x2 * x2
    x5 = x4 * x01
    env_r[...] = jnp.where(d < CUT, 1.0 - 6.0 * x5 + 15.0 * x4 - 10.0 * x3, 0.0)


def _edge_embed(ef_pad, dist, We, be):
    return pl.pallas_call(
        _edge_embed_body, grid=(E // EB,),
        in_specs=[_rowspec(EB, 40), _rowspec(EB, 1),
                  _fullspec((40, HID)), _fullspec((1, HID))],
        out_specs=[_rowspec(EB, HID), _rowspec(EB, 1)],
        out_shape=[jax.ShapeDtypeStruct((E, HID), f32),
                   jax.ShapeDtypeStruct((E, 1), f32)],
    )(ef_pad, dist, We, be)


def _edge_layer_body(e_r, ksvs_r, qd_r, env_r,
                     Wee_r, bee_r, Woe_r, boe_r, g1_r, b1_r,
                     Wf1_r, bf1_r, Wf2_r, bf2_r, g2_r, b2_r,
                     e2_r, contrib_r, arep_r):
    e = e_r[...]
    ks = ksvs_r[:, :HID]
    vs = ksvs_r[:, HID:]
    qd = qd_r[...]
    ee = _dot(e, Wee_r[...]) + bee_r[...]
    score = ks * qd * (0.25 * ee)
    s = jnp.clip(_dot(score, _sel_hd()), -5.0, 5.0)
    a = jnp.exp(s) * env_r[...]
    arep = _dot(a, _sel_dh())
    contrib_r[...] = arep * vs
    arep_r[...] = arep
    eh = e + _dot(score, Woe_r[...]) + boe_r[...]
    eh = _lnk(eh, g1_r[...], b1_r[...])
    h = jnp.maximum(_dot(eh, Wf1_r[...]) + bf1_r[...], 0.0)
    e2_r[...] = _lnk(eh + _dot(h, Wf2_r[...]) + bf2_r[...], g2_r[...], b2_r[...])


def _edge_layer(e, ksvs, qd, env, Wee, bee, Woe, boe, g1, b1,
                Wf1, bf1, Wf2, bf2, g2, b2):
    return pl.pallas_call(
        _edge_layer_body, grid=(E // EB,),
        in_specs=[_rowspec(EB, HID), _rowspec(EB, 2 * HID), _rowspec(EB, HID),
                  _rowspec(EB, 1),
                  _fullspec((HID, HID)), _fullspec((1, HID)),
                  _fullspec((HID, HID)), _fullspec((1, HID)),
                  _fullspec((1, HID)), _fullspec((1, HID)),
                  _fullspec((HID, FFN)), _fullspec((1, FFN)),
                  _fullspec((FFN, HID)), _fullspec((1, HID)),
                  _fullspec((1, HID)), _fullspec((1, HID))],
        out_specs=[_rowspec(EB, HID), _rowspec(EB, HID), _rowspec(EB, HID)],
        out_shape=[jax.ShapeDtypeStruct((E, HID), f32),
                   jax.ShapeDtypeStruct((E, HID), f32),
                   jax.ShapeDtypeStruct((E, HID), f32)],
    )(e, ksvs, qd, env, Wee, bee, Woe, boe, g1, b1, Wf1, bf1, Wf2, bf2, g2, b2)


def _node_common(v_r, aggp_r, denp_r, Wo_r, bo_r, g1_r, b1_r,
                 Wf1_r, bf1_r, Wf2_r, bf2_r, g2_r, b2_r):
    agg = aggp_r[0] + aggp_r[1]
    denr = denp_r[0] + denp_r[1] + 1e-6
    v_att = agg / denr
    vh = v_r[...] + _dot(v_att, Wo_r[...]) + bo_r[...]
    vh = _lnk(vh, g1_r[...], b1_r[...])
    h = jnp.maximum(_dot(vh, Wf1_r[...]) + bf1_r[...], 0.0)
    return _lnk(vh + _dot(h, Wf2_r[...]) + bf2_r[...], g2_r[...], b2_r[...])


def _node_layer_body(v_r, aggp_r, denp_r, Wo_r, bo_r, g1_r, b1_r,
                     Wf1_r, bf1_r, Wf2_r, bf2_r, g2_r, b2_r,
                     Wq_r, bq_r, Wk_r, bk_r, Wvv_r, bvv_r,
                     v2_r, q_r, kv_r):
    v2 = _node_common(v_r, aggp_r, denp_r, Wo_r, bo_r, g1_r, b1_r,
                      Wf1_r, bf1_r, Wf2_r, bf2_r, g2_r, b2_r)
    v2_r[...] = v2
    q_r[...] = _dot(v2, Wq_r[...]) + bq_r[...]
    k = _dot(v2, Wk_r[...]) + bk_r[...]
    val = _dot(v2, Wvv_r[...]) + bvv_r[...]
    kv_r[...] = jnp.concatenate([k, val], axis=1)


def _node_layer(v, aggp, denp, Wo, bo, g1, b1, Wf1, bf1, Wf2, bf2, g2, b2,
                Wq, bq, Wk, bk, Wvv, bvv):
    wspecs = [_fullspec((HID, HID)), _fullspec((1, HID)),
              _fullspec((1, HID)), _fullspec((1, HID)),
              _fullspec((HID, FFN)), _fullspec((1, FFN)),
              _fullspec((FFN, HID)), _fullspec((1, HID)),
              _fullspec((1, HID)), _fullspec((1, HID))]
    qkvspecs = [_fullspec((HID, HID)), _fullspec((1, HID)),
                _fullspec((HID, HID)), _fullspec((1, HID)),
                _fullspec((HID, HID)), _fullspec((1, HID))]
    return pl.pallas_call(
        _node_layer_body, grid=(N // NB,),
        in_specs=[_rowspec(NB, HID),
                  pl.BlockSpec((2, NB, HID), lambda i: (0, i, 0)),
                  pl.BlockSpec((2, NB, HID), lambda i: (0, i, 0))]
                 + wspecs + qkvspecs,
        out_specs=[_rowspec(NB, HID), _rowspec(NB, HID), _rowspec(NB, 2 * HID)],
        out_shape=[jax.ShapeDtypeStruct((N, HID), f32),
                   jax.ShapeDtypeStruct((N, HID), f32),
                   jax.ShapeDtypeStruct((N, 2 * HID), f32)],
    )(v, aggp, denp, Wo, bo, g1, b1, Wf1, bf1, Wf2, bf2, g2, b2,
      Wq, bq, Wk, bk, Wvv, bvv)


def _node_final_body(v_r, aggp_r, denp_r, nf0_r, rad_r,
                     Wo_r, bo_r, g1_r, b1_r, Wf1_r, bf1_r, Wf2_r, bf2_r,
                     g2_r, b2_r, vfin_r, z_r):
    v2 = _node_common(v_r, aggp_r, denp_r, Wo_r, bo_r, g1_r, b1_r,
                      Wf1_r, bf1_r, Wf2_r, bf2_r, g2_r, b2_r)
    vsc = v2 * nf0_r[...]
    vfin_r[:, :HID] = vsc
    lane = lax.broadcasted_iota(jnp.int32, (1, HID), 1)
    vfin_r[:, HID:] = jnp.where(lane == 0, rad_r[...], 0.0)

    @pl.when(pl.program_id(0) == 0)
    def _():
        z_r[...] = jnp.zeros_like(z_r)

    z_r[...] += jnp.sum(vsc, axis=0, keepdims=True)


def _node_final(v, aggp, denp, nf0, rad, Wo, bo, g1, b1, Wf1, bf1, Wf2, bf2,
                g2, b2):
    wspecs = [_fullspec((HID, HID)), _fullspec((1, HID)),
              _fullspec((1, HID)), _fullspec((1, HID)),
              _fullspec((HID, FFN)), _fullspec((1, FFN)),
              _fullspec((FFN, HID)), _fullspec((1, HID)),
              _fullspec((1, HID)), _fullspec((1, HID))]
    return pl.pallas_call(
        _node_final_body, grid=(N // NB,),
        in_specs=[_rowspec(NB, HID),
                  pl.BlockSpec((2, NB, HID), lambda i: (0, i, 0)),
                  pl.BlockSpec((2, NB, HID), lambda i: (0, i, 0)),
                  _rowspec(NB, 1), _rowspec(NB, 1)] + wspecs,
        out_specs=[_rowspec(NB, 2 * HID), _fullspec((1, HID))],
        out_shape=[jax.ShapeDtypeStruct((N, 2 * HID), f32),
                   jax.ShapeDtypeStruct((1, HID), f32)],
    )(v, aggp, denp, nf0, rad, Wo, bo, g1, b1, Wf1, bf1, Wf2, bf2, g2, b2)


def _sigmoid(x):
    return 1.0 / (1.0 + jnp.exp(-x))


def _readout_body(xs_r, xd_r, d_r, it_r, ef0_r,
                  W21_r, b21_r, W22_r, b22_r, W31_r, b31_r, W32_r, b32_r,
                  y_r):
    x = xs_r[:, :HID] + xd_r[:, :HID]
    rsum = xs_r[:, HID:HID + 1] + xd_r[:, HID:HID + 1]
    dist = d_r[...]
    d = dist - rsum
    t = d * 1.25
    V0 = -0.045 * jnp.exp(-(t * t))
    V1 = 0.8 * jnp.where(d < 0, d * d, 0.0)
    V2 = -0.035 * (jnp.where((d > 0) & (d < 2.5), -0.4 * (d - 2.5), 0.0)
                   + jnp.where(d <= 0, 1.0, 0.0))
    V3 = -0.6 * (jnp.where((d > -0.6) & (d < 0), (-5.0 / 3.0) * d, 0.0)
                 + jnp.where(d <= -0.6, 1.0, 0.0))
    mask = jnp.where(dist < CUT, 1.0, 0.0) * ef0_r[...]
    w2 = _sigmoid(_dot(jnp.maximum(_dot(x, W21_r[...]) + b21_r[...], 0.0),
                       W22_r[...]) + b22_r[...]) + 0.5
    w3 = _sigmoid(_dot(jnp.maximum(_dot(x, W31_r[...]) + b31_r[...], 0.0),
                       W32_r[...]) + b32_r[...]) + 0.5
    it1 = it_r[:, 1:2]
    it2 = it_r[:, 2:3]
    t2 = (w2[:, 0:1] * V0 + w2[:, 1:2] * V1 + it1 * w2[:, 2:3] * V2
          + it2 * w2[:, 3:4] * V3)
    t3 = (w3[:, 0:1] * V0 + w3[:, 1:2] * V1 + it1 * w3[:, 2:3] * V2
          + it2 * w3[:, 3:4] * V3)
    p2 = jnp.sum(mask * t2) * 0.5
    p3 = jnp.sum(mask * t3) * 0.5

    @pl.when(pl.program_id(0) == 0)
    def _():
        y_r[...] = jnp.zeros_like(y_r)

    lane = lax.broadcasted_iota(jnp.int32, (1, HID), 1)
    y_r[...] += jnp.where(lane == 0, p2, 0.0) + jnp.where(lane == 1, p3, 0.0)


def _readout(xs, xd, dist, itp, ef0, W21, b21, W22, b22, W31, b31, W32, b32):
    return pl.pallas_call(
        _readout_body, grid=(E // EB,),
        in_specs=[_rowspec(EB, 2 * HID), _rowspec(EB, 2 * HID), _rowspec(EB, 1),
                  _rowspec(EB, 8), _rowspec(EB, 1),
                  _fullspec((HID, HID)), _fullspec((1, HID)),
                  _fullspec((HID, 8)), _fullspec((1, 8)),
                  _fullspec((HID, HID)), _fullspec((1, HID)),
                  _fullspec((HID, 8)), _fullspec((1, 8))],
        out_specs=_fullspec((1, HID)),
        out_shape=jax.ShapeDtypeStruct((1, HID), f32),
    )(xs, xd, dist, itp, ef0, W21, b21, W22, b22, W31, b31, W32, b32)


def _head_body(z_r, y_r, nrot_r, W11_r, b11_r, W12_r, b12_r,
               W41_r, b41_r, W42_r, b42_r, out_r):
    z = z_r[...]
    y1 = (_dot(jnp.maximum(_dot(z, W11_r[...]) + b11_r[...], 0.0),
               W12_r[...]) + b12_r[...])[0, 0]
    w4 = _sigmoid((_dot(jnp.maximum(_dot(z, W41_r[...]) + b41_r[...], 0.0),
                        W42_r[...]) + b42_r[...])[0, 0]) + 0.5
    y2 = y_r[0, 0]
    y3 = y_r[0, 1] / (1.0 + w4 * 0.05846 * nrot_r[0, 0])
    lane = lax.broadcasted_iota(jnp.int32, (1, HID), 1)
    out_r[...] = (jnp.where(lane == 0, y1, 0.0)
                  + jnp.where(lane == 1, y2, 0.0)
                  + jnp.where(lane == 2, y3, 0.0))


def _head(z, y, nrot, W11, b11, W12, b12, W41, b41, W42, b42):
    return pl.pallas_call(
        _head_body,
        out_shape=jax.ShapeDtypeStruct((1, HID), f32),
    )(z, y, nrot, W11, b11, W12, b12, W41, b41, W42, b42)


def kernel(node_feature, edge_feature, vdw_radii, distance, interaction_type,
           edge_index, n_rot, params):
    p = params
    src = edge_index[0].astype(jnp.int32)
    dst = edge_index[1].astype(jnp.int32)
    nf_pad = jnp.pad(node_feature, ((0, 0), (0, 1)))
    ef_pad = jnp.pad(edge_feature, ((0, 0), (0, 1)))
    Wv = jnp.pad(p['Wv_emb'], ((0, 1), (0, 0)))
    We = jnp.pad(p['We_emb'], ((0, 1), (0, 0)))
    dist = distance[:, None]
    itp = jnp.pad(interaction_type, ((0, 0), (0, 5)))
    ef0 = edge_feature[:, 0:1]
    nf0 = node_feature[:, 0:1]
    rad = vdw_radii[:, None]
    nrot = jnp.asarray(n_rot, f32).reshape(1, 1)

    def row(x):
        return x.reshape(1, -1)

    v, q_t, kv_t = _node_embed(nf_pad, Wv, row(p['bv_emb']),
                               p['Wq'][0], row(p['bq'][0]),
                               p['Wk'][0], row(p['bk'][0]),
                               p['Wvv'][0], row(p['bvv'][0]))
    e, env = _edge_embed(ef_pad, dist, We, row(p['be_emb']))

    vfin = None
    z = None
    for l in range(NLAYERS):
        ksvs, qd = _gather_layer(kv_t, q_t, src, dst)
        e, contrib, arep = _edge_layer(
            e, ksvs, qd, env,
            p['Wee'][l], row(p['bee'][l]), p['Woe'][l], row(p['boe'][l]),
            row(p['g1e'][l]), row(p['b1e'][l]),
            p['Wf1e'][l], row(p['bf1e'][l]), p['Wf2e'][l], row(p['bf2e'][l]),
            row(p['g2e'][l]), row(p['b2e'][l]))
        aggf, denf = _scatter(contrib, arep, dst)
        aggp = aggf.reshape(2, NPAD, HID)
        denp = denf.reshape(2, NPAD, HID)
        nw = (p['Wo'][l], row(p['bo'][l]), row(p['g1v'][l]), row(p['b1v'][l]),
              p['Wf1v'][l], row(p['bf1v'][l]), p['Wf2v'][l], row(p['bf2v'][l]),
              row(p['g2v'][l]), row(p['b2v'][l]))
        if l < NLAYERS - 1:
            v, q_t, kv_t = _node_layer(v, aggp, denp, *nw,
                                       p['Wq'][l + 1], row(p['bq'][l + 1]),
                                       p['Wk'][l + 1], row(p['bk'][l + 1]),
                                       p['Wvv'][l + 1], row(p['bvv'][l + 1]))
        else:
            vfin, z = _node_final(v, aggp, denp, nf0, rad, *nw)

    xs, xd = _gather_fin(vfin, vfin, src, dst)
    y23 = _readout(xs, xd, dist, itp, ef0,
                   p['r2_W1'], row(p['r2_b1']),
                   jnp.pad(p['r2_W2'], ((0, 0), (0, 4))),
                   row(jnp.pad(p['r2_b2'], (0, 4))),
                   p['r3_W1'], row(p['r3_b1']),
                   jnp.pad(p['r3_W2'], ((0, 0), (0, 4))),
                   row(jnp.pad(p['r3_b2'], (0, 4))))
    out = _head(z, y23, nrot,
                p['r1_W1'], row(p['r1_b1']),
                jnp.pad(p['r1_W2'], ((0, 0), (0, 7))),
                row(jnp.pad(p['r1_b2'], (0, 7))),
                p['r4_W1'], row(p['r4_b1']),
                jnp.pad(p['r4_W2'], ((0, 0), (0, 7))),
                row(jnp.pad(p['r4_b2'], (0, 7))))
    return out[0, :3]


# speedup vs baseline: 31.0893x; 1.1416x over previous
import functools

import jax
import jax.numpy as jnp
from jax import lax
from jax.experimental import pallas as pl
from jax.experimental.pallas import tpu as pltpu
from jax.experimental.pallas import tpu_sc as plsc

N = 10000
E = 160000
HID = 128
HEADS = 8
DH = HID // HEADS
NLAYERS = 10
FFN = 2 * HID
CUT = 6.5
f32 = jnp.float32

NC = 2
NS = 16
NW = NC * NS
PER_W = E // NW
CH = 104
NFULL = PER_W // CH
NPAIR = NFULL // 2
TAIL = PER_W - NFULL * CH
DCH = 128
NPAD = 10240
TROWS = NPAD // NS
ZR = 128

EB = 2000
NB = 1000

_PREC = jax.lax.Precision.DEFAULT


def _dot(a, b):
    return jnp.dot(a, b, preferred_element_type=f32, precision=_PREC)


def _lnk(x, g, b):
    m = jnp.mean(x, axis=-1, keepdims=True)
    v = jnp.mean((x - m) * (x - m), axis=-1, keepdims=True)
    return (x - m) * lax.rsqrt(v + 1e-5) * g + b


def _sel_hd():
    r = lax.broadcasted_iota(jnp.int32, (HID, HEADS), 0) // DH
    c = lax.broadcasted_iota(jnp.int32, (HID, HEADS), 1)
    return (r == c).astype(f32)


def _sel_dh():
    r = lax.broadcasted_iota(jnp.int32, (HEADS, HID), 0)
    c = lax.broadcasted_iota(jnp.int32, (HEADS, HID), 1) // DH
    return (r == c).astype(f32)


def _rowspec(b, w):
    return pl.BlockSpec((b, w), lambda i: (i, 0))


def _fullspec(shape):
    return pl.BlockSpec(shape, lambda i: tuple(0 for _ in shape))


@functools.lru_cache(maxsize=None)
def _make_gather(wa, wb):
    mesh = plsc.VectorSubcoreMesh(core_axis_name="c", subcore_axis_name="s")

    @functools.partial(
        pl.kernel, mesh=mesh,
        out_type=(jax.ShapeDtypeStruct((E, wa), f32),
                  jax.ShapeDtypeStruct((E, wb), f32)),
        scratch_types=(
            pltpu.VMEM((2, CH), jnp.int32), pltpu.VMEM((2, CH), jnp.int32),
            pltpu.VMEM((TAIL,), jnp.int32), pltpu.VMEM((TAIL,), jnp.int32),
            pltpu.VMEM((2, CH, wa), f32), pltpu.VMEM((2, CH, wb), f32),
            pltpu.VMEM((TAIL, wa), f32), pltpu.VMEM((TAIL, wb), f32),
            pltpu.SemaphoreType.DMA, pltpu.SemaphoreType.DMA,
            pltpu.SemaphoreType.DMA, pltpu.SemaphoreType.DMA,
            pltpu.SemaphoreType.DMA, pltpu.SemaphoreType.DMA,
        ),
    )
    def gk(tabA, tabB, idxA, idxB, outA, outB,
           ia, ib, ta, tb, ra, rb, tra, trb,
           sga, sgb, swa0, swb0, swa1, swb1):
        wid = lax.axis_index("s") * NC + lax.axis_index("c")
        base = wid * PER_W
        sw = ((swa0, swb0), (swa1, swb1))

        def pair(g, carry):
            @pl.when(g > 0)
            def _():
                for b in range(2):
                    pltpu.make_async_copy(ra.at[b], outA.at[pl.ds(0, CH)],
                                          sw[b][0]).wait()
                    pltpu.make_async_copy(rb.at[b], outB.at[pl.ds(0, CH)],
                                          sw[b][1]).wait()

            copies = []
            for b in range(2):
                off = base + (g * 2 + b) * CH
                pltpu.sync_copy(idxA.at[pl.ds(off, CH)], ia.at[b])
                pltpu.sync_copy(idxB.at[pl.ds(off, CH)], ib.at[b])
                copies.append(
                    (pltpu.async_copy(tabA.at[ia.at[b]], ra.at[b], sga),
                     pltpu.async_copy(tabB.at[ib.at[b]], rb.at[b], sgb)))
            for b in range(2):
                off = base + (g * 2 + b) * CH
                ca, cb = copies[b]
                ca.wait()
                cb.wait()
                pltpu.async_copy(ra.at[b], outA.at[pl.ds(off, CH)], sw[b][0])
                pltpu.async_copy(rb.at[b], outB.at[pl.ds(off, CH)], sw[b][1])
            return carry

        lax.fori_loop(0, NPAIR, pair, 0)
        for b in range(2):
            pltpu.make_async_copy(ra.at[b], outA.at[pl.ds(0, CH)],
                                  sw[b][0]).wait()
            pltpu.make_async_copy(rb.at[b], outB.at[pl.ds(0, CH)],
                                  sw[b][1]).wait()
        off = base + NFULL * CH
        pltpu.sync_copy(idxA.at[pl.ds(off, TAIL)], ta)
        pltpu.sync_copy(idxB.at[pl.ds(off, TAIL)], tb)
        ca = pltpu.async_copy(tabA.at[ta], tra, sga)
        cb = pltpu.async_copy(tabB.at[tb], trb, sgb)
        ca.wait()
        cb.wait()
        pltpu.sync_copy(tra, outA.at[pl.ds(off, TAIL)])
        pltpu.sync_copy(trb, outB.at[pl.ds(off, TAIL)])

    return gk


def _gather_layer(kv_t, q_t, src, dst):
    return _make_gather(2 * HID, HID)(kv_t, q_t, src, dst)


def _gather_fin(vfin_a, vfin_b, src, dst):
    return _make_gather(2 * HID, 2 * HID)(vfin_a, vfin_b, src, dst)


@functools.lru_cache(maxsize=None)
def _make_scatter():
    mesh = plsc.VectorSubcoreMesh(core_axis_name="c", subcore_axis_name="s")

    @functools.partial(
        pl.kernel, mesh=mesh,
        out_type=(jax.ShapeDtypeStruct((2 * NPAD, HID), f32),
                  jax.ShapeDtypeStruct((2 * NPAD, HID), f32)),
        scratch_types=(
            pltpu.VMEM((2, CH), jnp.int32),
            pltpu.VMEM((2, CH, HID), f32),
            pltpu.VMEM((TAIL,), jnp.int32),
            pltpu.VMEM((TAIL, HID), f32),
            pltpu.VMEM((ZR, HID), f32),
            pltpu.VMEM_SHARED((NPAD, HID), f32),
            pltpu.SemaphoreType.DMA, pltpu.SemaphoreType.DMA,
        ),
    )
    def sck(contrib_h, arep_h, dst_h, agg_out, den_out,
            idx_v, rows_v, idx_t, rows_t, zb, acc_sh, sl0, sl1):
        c = lax.axis_index("c")
        s = lax.axis_index("s")
        zeros16 = jnp.zeros((16,), f32)

        def zero_zb():
            def zrow(r, carry):
                def zcol(j, cc):
                    zb[r, pl.ds(j * 16, 16)] = zeros16
                    return cc
                return lax.fori_loop(0, HID // 16, zcol, carry)

            lax.fori_loop(0, ZR, zrow, 0)

        zero_zb()

        r0 = s * TROWS
        wid = s * NC + c
        base = wid * PER_W
        sl = (sl0, sl1)

        def zero_acc():
            for b in range(TROWS // ZR):
                pltpu.sync_copy(zb, acc_sh.at[pl.ds(r0 + b * ZR, ZR)])

        def scatter_pass(src_h, out_h):
            def pair(g, carry):
                copies = []
                for b in range(2):
                    off = base + (g * 2 + b) * CH
                    pltpu.sync_copy(dst_h.at[pl.ds(off, CH)], idx_v.at[b])
                    copies.append(pltpu.async_copy(
                        src_h.at[pl.ds(off, CH)], rows_v.at[b], sl[b]))
                for b in range(2):
                    copies[b].wait()
                    pltpu.sync_copy(rows_v.at[b], acc_sh.at[idx_v.at[b]],
                                    add=True)
                return carry

            lax.fori_loop(0, NPAIR, pair, 0)
            off = base + NFULL * CH
            pltpu.sync_copy(dst_h.at[pl.ds(off, TAIL)], idx_t)
            pltpu.sync_copy(src_h.at[pl.ds(off, TAIL)], rows_t)
            pltpu.sync_copy(rows_t, acc_sh.at[idx_t], add=True)
            plsc.subcore_barrier()
            for b in range(TROWS // DCH):
                pltpu.sync_copy(acc_sh.at[pl.ds(r0 + b * DCH, DCH)], zb)
                pltpu.sync_copy(zb,
                                out_h.at[pl.ds(c * NPAD + r0 + b * DCH, DCH)])
            plsc.subcore_barrier()

        zero_acc()
        plsc.subcore_barrier()
        scatter_pass(contrib_h, agg_out)
        zero_zb()
        zero_acc()
        plsc.subcore_barrier()
        scatter_pass(arep_h, den_out)

    return sck


def _scatter(contrib, arep, dst):
    return _make_scatter()(contrib, arep, dst)


def _node_embed_body(nf_r, Wv_r, bv_r, Wq_r, bq_r, Wk_r, bk_r, Wvv_r, bvv_r,
                     v_r, q_r, kv_r):
    v = _dot(nf_r[...], Wv_r[...]) + bv_r[...]
    v_r[...] = v
    q_r[...] = _dot(v, Wq_r[...]) + bq_r[...]
    k = _dot(v, Wk_r[...]) + bk_r[...]
    val = _dot(v, Wvv_r[...]) + bvv_r[...]
    kv_r[...] = jnp.concatenate([k, val], axis=1)


def _node_embed(nf_pad, Wv, bv, Wq, bq, Wk, bk, Wvv, bvv):
    return pl.pallas_call(
        _node_embed_body, grid=(N // NB,),
        in_specs=[_rowspec(NB, 40), _fullspec((40, HID)), _fullspec((1, HID)),
                  _fullspec((HID, HID)), _fullspec((1, HID)),
                  _fullspec((HID, HID)), _fullspec((1, HID)),
                  _fullspec((HID, HID)), _fullspec((1, HID))],
        out_specs=[_rowspec(NB, HID), _rowspec(NB, HID), _rowspec(NB, 2 * HID)],
        out_shape=[jax.ShapeDtypeStruct((N, HID), f32),
                   jax.ShapeDtypeStruct((N, HID), f32),
                   jax.ShapeDtypeStruct((N, 2 * HID), f32)],
    )(nf_pad, Wv, bv, Wq, bq, Wk, bk, Wvv, bvv)


def _edge_embed_body(ef_r, d_r, We_r, be_r, e_r, env_r):
    e_r[...] = _dot(ef_r[...], We_r[...]) + be_r[...]
    d = d_r[...]
    x01 = jnp.clip(d / CUT, 0.0, 1.0)
    x2 = x01 * x01
    x3 = x2 * x01
    x4 = x2 * x2
    x5 = x4 * x01
    env_r[...] = jnp.where(d < CUT, 1.0 - 6.0 * x5 + 15.0 * x4 - 10.0 * x3, 0.0)


def _edge_embed(ef_pad, dist, We, be):
    return pl.pallas_call(
        _edge_embed_body, grid=(E // EB,),
        in_specs=[_rowspec(EB, 40), _rowspec(EB, 1),
                  _fullspec((40, HID)), _fullspec((1, HID))],
        out_specs=[_rowspec(EB, HID), _rowspec(EB, 1)],
        out_shape=[jax.ShapeDtypeStruct((E, HID), f32),
                   jax.ShapeDtypeStruct((E, 1), f32)],
    )(ef_pad, dist, We, be)


def _edge_layer_body(e_r, ksvs_r, qd_r, env_r,
                     Wee_r, bee_r, Woe_r, boe_r, g1_r, b1_r,
                     Wf1_r, bf1_r, Wf2_r, bf2_r, g2_r, b2_r,
                     e2_r, contrib_r, arep_r):
    e = e_r[...]
    ks = ksvs_r[:, :HID]
    vs = ksvs_r[:, HID:]
    qd = qd_r[...]
    ee = _dot(e, Wee_r[...]) + bee_r[...]
    score = ks * qd * (0.25 * ee)
    s = jnp.clip(_dot(score, _sel_hd()), -5.0, 5.0)
    a = jnp.exp(s) * env_r[...]
    arep = _dot(a, _sel_dh())
    contrib_r[...] = arep * vs
    arep_r[...] = arep
    eh = e + _dot(score, Woe_r[...]) + boe_r[...]
    eh = _lnk(eh, g1_r[...], b1_r[...])
    h = jnp.maximum(_dot(eh, Wf1_r[...]) + bf1_r[...], 0.0)
    e2_r[...] = _lnk(eh + _dot(h, Wf2_r[...]) + bf2_r[...], g2_r[...], b2_r[...])


def _edge_layer(e, ksvs, qd, env, Wee, bee, Woe, boe, g1, b1,
                Wf1, bf1, Wf2, bf2, g2, b2):
    return pl.pallas_call(
        _edge_layer_body, grid=(E // EB,),
        in_specs=[_rowspec(EB, HID), _rowspec(EB, 2 * HID), _rowspec(EB, HID),
                  _rowspec(EB, 1),
                  _fullspec((HID, HID)), _fullspec((1, HID)),
                  _fullspec((HID, HID)), _fullspec((1, HID)),
                  _fullspec((1, HID)), _fullspec((1, HID)),
                  _fullspec((HID, FFN)), _fullspec((1, FFN)),
                  _fullspec((FFN, HID)), _fullspec((1, HID)),
                  _fullspec((1, HID)), _fullspec((1, HID))],
        out_specs=[_rowspec(EB, HID), _rowspec(EB, HID), _rowspec(EB, HID)],
        out_shape=[jax.ShapeDtypeStruct((E, HID), f32),
                   jax.ShapeDtypeStruct((E, HID), f32),
                   jax.ShapeDtypeStruct((E, HID), f32)],
    )(e, ksvs, qd, env, Wee, bee, Woe, boe, g1, b1, Wf1, bf1, Wf2, bf2, g2, b2)


def _node_common(v_r, aggp_r, denp_r, Wo_r, bo_r, g1_r, b1_r,
                 Wf1_r, bf1_r, Wf2_r, bf2_r, g2_r, b2_r):
    agg = aggp_r[0] + aggp_r[1]
    denr = denp_r[0] + denp_r[1] + 1e-6
    v_att = agg / denr
    vh = v_r[...] + _dot(v_att, Wo_r[...]) + bo_r[...]
    vh = _lnk(vh, g1_r[...], b1_r[...])
    h = jnp.maximum(_dot(vh, Wf1_r[...]) + bf1_r[...], 0.0)
    return _lnk(vh + _dot(h, Wf2_r[...]) + bf2_r[...], g2_r[...], b2_r[...])


def _node_layer_body(v_r, aggp_r, denp_r, Wo_r, bo_r, g1_r, b1_r,
                     Wf1_r, bf1_r, Wf2_r, bf2_r, g2_r, b2_r,
                     Wq_r, bq_r, Wk_r, bk_r, Wvv_r, bvv_r,
                     v2_r, q_r, kv_r):
    v2 = _node_common(v_r, aggp_r, denp_r, Wo_r, bo_r, g1_r, b1_r,
                      Wf1_r, bf1_r, Wf2_r, bf2_r, g2_r, b2_r)
    v2_r[...] = v2
    q_r[...] = _dot(v2, Wq_r[...]) + bq_r[...]
    k = _dot(v2, Wk_r[...]) + bk_r[...]
    val = _dot(v2, Wvv_r[...]) + bvv_r[...]
    kv_r[...] = jnp.concatenate([k, val], axis=1)


def _node_layer(v, aggp, denp, Wo, bo, g1, b1, Wf1, bf1, Wf2, bf2, g2, b2,
                Wq, bq, Wk, bk, Wvv, bvv):
    wspecs = [_fullspec((HID, HID)), _fullspec((1, HID)),
              _fullspec((1, HID)), _fullspec((1, HID)),
              _fullspec((HID, FFN)), _fullspec((1, FFN)),
              _fullspec((FFN, HID)), _fullspec((1, HID)),
              _fullspec((1, HID)), _fullspec((1, HID))]
    qkvspecs = [_fullspec((HID, HID)), _fullspec((1, HID)),
                _fullspec((HID, HID)), _fullspec((1, HID)),
                _fullspec((HID, HID)), _fullspec((1, HID))]
    return pl.pallas_call(
        _node_layer_body, grid=(N // NB,),
        in_specs=[_rowspec(NB, HID),
                  pl.BlockSpec((2, NB, HID), lambda i: (0, i, 0)),
                  pl.BlockSpec((2, NB, HID), lambda i: (0, i, 0))]
                 + wspecs + qkvspecs,
        out_specs=[_rowspec(NB, HID), _rowspec(NB, HID), _rowspec(NB, 2 * HID)],
        out_shape=[jax.ShapeDtypeStruct((N, HID), f32),
                   jax.ShapeDtypeStruct((N, HID), f32),
                   jax.ShapeDtypeStruct((N, 2 * HID), f32)],
    )(v, aggp, denp, Wo, bo, g1, b1, Wf1, bf1, Wf2, bf2, g2, b2,
      Wq, bq, Wk, bk, Wvv, bvv)


def _node_final_body(v_r, aggp_r, denp_r, nf0_r, rad_r,
                     Wo_r, bo_r, g1_r, b1_r, Wf1_r, bf1_r, Wf2_r, bf2_r,
                     g2_r, b2_r, vfin_r, z_r):
    v2 = _node_common(v_r, aggp_r, denp_r, Wo_r, bo_r, g1_r, b1_r,
                      Wf1_r, bf1_r, Wf2_r, bf2_r, g2_r, b2_r)
    vsc = v2 * nf0_r[...]
    vfin_r[:, :HID] = vsc
    lane = lax.broadcasted_iota(jnp.int32, (1, HID), 1)
    vfin_r[:, HID:] = jnp.where(lane == 0, rad_r[...], 0.0)

    @pl.when(pl.program_id(0) == 0)
    def _():
        z_r[...] = jnp.zeros_like(z_r)

    z_r[...] += jnp.sum(vsc, axis=0, keepdims=True)


def _node_final(v, aggp, denp, nf0, rad, Wo, bo, g1, b1, Wf1, bf1, Wf2, bf2,
                g2, b2):
    wspecs = [_fullspec((HID, HID)), _fullspec((1, HID)),
              _fullspec((1, HID)), _fullspec((1, HID)),
              _fullspec((HID, FFN)), _fullspec((1, FFN)),
              _fullspec((FFN, HID)), _fullspec((1, HID)),
              _fullspec((1, HID)), _fullspec((1, HID))]
    return pl.pallas_call(
        _node_final_body, grid=(N // NB,),
        in_specs=[_rowspec(NB, HID),
                  pl.BlockSpec((2, NB, HID), lambda i: (0, i, 0)),
                  pl.BlockSpec((2, NB, HID), lambda i: (0, i, 0)),
                  _rowspec(NB, 1), _rowspec(NB, 1)] + wspecs,
        out_specs=[_rowspec(NB, 2 * HID), _fullspec((1, HID))],
        out_shape=[jax.ShapeDtypeStruct((N, 2 * HID), f32),
                   jax.ShapeDtypeStruct((1, HID), f32)],
    )(v, aggp, denp, nf0, rad, Wo, bo, g1, b1, Wf1, bf1, Wf2, bf2, g2, b2)


def _sigmoid(x):
    return 1.0 / (1.0 + jnp.exp(-x))


def _readout_body(xs_r, xd_r, d_r, it_r, ef0_r,
                  W21_r, b21_r, W22_r, b22_r, W31_r, b31_r, W32_r, b32_r,
                  y_r):
    x = xs_r[:, :HID] + xd_r[:, :HID]
    rsum = xs_r[:, HID:HID + 1] + xd_r[:, HID:HID + 1]
    dist = d_r[...]
    d = dist - rsum
    t = d * 1.25
    V0 = -0.045 * jnp.exp(-(t * t))
    V1 = 0.8 * jnp.where(d < 0, d * d, 0.0)
    V2 = -0.035 * (jnp.where((d > 0) & (d < 2.5), -0.4 * (d - 2.5), 0.0)
                   + jnp.where(d <= 0, 1.0, 0.0))
    V3 = -0.6 * (jnp.where((d > -0.6) & (d < 0), (-5.0 / 3.0) * d, 0.0)
                 + jnp.where(d <= -0.6, 1.0, 0.0))
    mask = jnp.where(dist < CUT, 1.0, 0.0) * ef0_r[...]
    w2 = _sigmoid(_dot(jnp.maximum(_dot(x, W21_r[...]) + b21_r[...], 0.0),
                       W22_r[...]) + b22_r[...]) + 0.5
    w3 = _sigmoid(_dot(jnp.maximum(_dot(x, W31_r[...]) + b31_r[...], 0.0),
                       W32_r[...]) + b32_r[...]) + 0.5
    it1 = it_r[:, 1:2]
    it2 = it_r[:, 2:3]
    t2 = (w2[:, 0:1] * V0 + w2[:, 1:2] * V1 + it1 * w2[:, 2:3] * V2
          + it2 * w2[:, 3:4] * V3)
    t3 = (w3[:, 0:1] * V0 + w3[:, 1:2] * V1 + it1 * w3[:, 2:3] * V2
          + it2 * w3[:, 3:4] * V3)
    p2 = jnp.sum(mask * t2) * 0.5
    p3 = jnp.sum(mask * t3) * 0.5

    @pl.when(pl.program_id(0) == 0)
    def _():
        y_r[...] = jnp.zeros_like(y_r)

    lane = lax.broadcasted_iota(jnp.int32, (1, HID), 1)
    y_r[...] += jnp.where(lane == 0, p2, 0.0) + jnp.where(lane == 1, p3, 0.0)


def _readout(xs, xd, dist, itp, ef0, W21, b21, W22, b22, W31, b31, W32, b32):
    return pl.pallas_call(
        _readout_body, grid=(E // EB,),
        in_specs=[_rowspec(EB, 2 * HID), _rowspec(EB, 2 * HID), _rowspec(EB, 1),
                  _rowspec(EB, 8), _rowspec(EB, 1),
                  _fullspec((HID, HID)), _fullspec((1, HID)),
                  _fullspec((HID, 8)), _fullspec((1, 8)),
                  _fullspec((HID, HID)), _fullspec((1, HID)),
                  _fullspec((HID, 8)), _fullspec((1, 8))],
        out_specs=_fullspec((1, HID)),
        out_shape=jax.ShapeDtypeStruct((1, HID), f32),
    )(xs, xd, dist, itp, ef0, W21, b21, W22, b22, W31, b31, W32, b32)


def _head_body(z_r, y_r, nrot_r, W11_r, b11_r, W12_r, b12_r,
               W41_r, b41_r, W42_r, b42_r, out_r):
    z = z_r[...]
    y1 = (_dot(jnp.maximum(_dot(z, W11_r[...]) + b11_r[...], 0.0),
               W12_r[...]) + b12_r[...])[0, 0]
    w4 = _sigmoid((_dot(jnp.maximum(_dot(z, W41_r[...]) + b41_r[...], 0.0),
                        W42_r[...]) + b42_r[...])[0, 0]) + 0.5
    y2 = y_r[0, 0]
    y3 = y_r[0, 1] / (1.0 + w4 * 0.05846 * nrot_r[0, 0])
    lane = lax.broadcasted_iota(jnp.int32, (1, HID), 1)
    out_r[...] = (jnp.where(lane == 0, y1, 0.0)
                  + jnp.where(lane == 1, y2, 0.0)
                  + jnp.where(lane == 2, y3, 0.0))


def _head(z, y, nrot, W11, b11, W12, b12, W41, b41, W42, b42):
    return pl.pallas_call(
        _head_body,
        out_shape=jax.ShapeDtypeStruct((1, HID), f32),
    )(z, y, nrot, W11, b11, W12, b12, W41, b41, W42, b42)


def kernel(node_feature, edge_feature, vdw_radii, distance, interaction_type,
           edge_index, n_rot, params):
    p = params
    src = edge_index[0].astype(jnp.int32)
    dst = edge_index[1].astype(jnp.int32)
    nf_pad = jnp.pad(node_feature, ((0, 0), (0, 1)))
    ef_pad = jnp.pad(edge_feature, ((0, 0), (0, 1)))
    Wv = jnp.pad(p['Wv_emb'], ((0, 1), (0, 0)))
    We = jnp.pad(p['We_emb'], ((0, 1), (0, 0)))
    dist = distance[:, None]
    itp = jnp.pad(interaction_type, ((0, 0), (0, 5)))
    ef0 = edge_feature[:, 0:1]
    nf0 = node_feature[:, 0:1]
    rad = vdw_radii[:, None]
    nrot = jnp.asarray(n_rot, f32).reshape(1, 1)

    def row(x):
        return x.reshape(1, -1)

    v, q_t, kv_t = _node_embed(nf_pad, Wv, row(p['bv_emb']),
                               p['Wq'][0], row(p['bq'][0]),
                               p['Wk'][0], row(p['bk'][0]),
                               p['Wvv'][0], row(p['bvv'][0]))
    e, env = _edge_embed(ef_pad, dist, We, row(p['be_emb']))

    vfin = None
    z = None
    for l in range(NLAYERS):
        ksvs, qd = _gather_layer(kv_t, q_t, src, dst)
        e, contrib, arep = _edge_layer(
            e, ksvs, qd, env,
            p['Wee'][l], row(p['bee'][l]), p['Woe'][l], row(p['boe'][l]),
            row(p['g1e'][l]), row(p['b1e'][l]),
            p['Wf1e'][l], row(p['bf1e'][l]), p['Wf2e'][l], row(p['bf2e'][l]),
            row(p['g2e'][l]), row(p['b2e'][l]))
        aggf, denf = _scatter(contrib, arep, dst)
        aggp = aggf.reshape(2, NPAD, HID)
        denp = denf.reshape(2, NPAD, HID)
        nw = (p['Wo'][l], row(p['bo'][l]), row(p['g1v'][l]), row(p['b1v'][l]),
              p['Wf1v'][l], row(p['bf1v'][l]), p['Wf2v'][l], row(p['bf2v'][l]),
              row(p['g2v'][l]), row(p['b2v'][l]))
        if l < NLAYERS - 1:
            v, q_t, kv_t = _node_layer(v, aggp, denp, *nw,
                                       p['Wq'][l + 1], row(p['bq'][l + 1]),
                                       p['Wk'][l + 1], row(p['bk'][l + 1]),
                                       p['Wvv'][l + 1], row(p['bvv'][l + 1]))
        else:
            vfin, z = _node_final(v, aggp, denp, nf0, rad, *nw)

    xs, xd = _gather_fin(vfin, vfin, src, dst)
    y23 = _readout(xs, xd, dist, itp, ef0,
                   p['r2_W1'], row(p['r2_b1']),
                   jnp.pad(p['r2_W2'], ((0, 0), (0, 4))),
                   row(jnp.pad(p['r2_b2'], (0, 4))),
                   p['r3_W1'], row(p['r3_b1']),
                   jnp.pad(p['r3_W2'], ((0, 0), (0, 4))),
                   row(jnp.pad(p['r3_b2'], (0, 4))))
    out = _head(z, y23, nrot,
                p['r1_W1'], row(p['r1_b1']),
                jnp.pad(p['r1_W2'], ((0, 0), (0, 7))),
                row(jnp.pad(p['r1_b2'], (0, 7))),
                p['r4_W1'], row(p['r4_b1']),
                jnp.pad(p['r4_W2'], ((0, 0), (0, 7))),
                row(jnp.pad(p['r4_b2'], (0, 7))))
    return out[0, :3]


# speedup vs baseline: 34.2330x; 1.1011x over previous
import functools

import jax
import jax.numpy as jnp
from jax import lax
from jax.experimental import pallas as pl
from jax.experimental.pallas import tpu as pltpu
from jax.experimental.pallas import tpu_sc as plsc

N = 10000
E = 160000
HID = 128
HEADS = 8
DH = HID // HEADS
NLAYERS = 10
FFN = 2 * HID
CUT = 6.5
f32 = jnp.float32

NC = 2
NS = 16
NW = NC * NS
PER_W = E // NW
CH = 104
NFULL = PER_W // CH
NPAIR = NFULL // 2
TAIL = PER_W - NFULL * CH
DCH = 128
NPAD = 10240
TROWS = NPAD // NS
ZR = 128

EB = 2000
NB = 1000

_PREC = jax.lax.Precision.DEFAULT


def _dot(a, b):
    return jnp.dot(a, b, preferred_element_type=f32, precision=_PREC)


def _lnk(x, g, b):
    m = jnp.mean(x, axis=-1, keepdims=True)
    v = jnp.mean((x - m) * (x - m), axis=-1, keepdims=True)
    return (x - m) * lax.rsqrt(v + 1e-5) * g + b


def _sel_hd():
    r = lax.broadcasted_iota(jnp.int32, (HID, HEADS), 0) // DH
    c = lax.broadcasted_iota(jnp.int32, (HID, HEADS), 1)
    return (r == c).astype(f32)


def _sel_dh():
    r = lax.broadcasted_iota(jnp.int32, (HEADS, HID), 0)
    c = lax.broadcasted_iota(jnp.int32, (HEADS, HID), 1) // DH
    return (r == c).astype(f32)


def _rowspec(b, w):
    return pl.BlockSpec((b, w), lambda i: (i, 0))


def _fullspec(shape):
    return pl.BlockSpec(shape, lambda i: tuple(0 for _ in shape))


@functools.lru_cache(maxsize=None)
def _make_gather(rsa, dta, rsb, dtb):
    mesh = plsc.VectorSubcoreMesh(core_axis_name="c", subcore_axis_name="s")

    @functools.partial(
        pl.kernel, mesh=mesh,
        out_type=(jax.ShapeDtypeStruct((E,) + rsa, dta),
                  jax.ShapeDtypeStruct((E,) + rsb, dtb)),
        scratch_types=(
            pltpu.VMEM((2, CH), jnp.int32), pltpu.VMEM((2, CH), jnp.int32),
            pltpu.VMEM((TAIL,), jnp.int32), pltpu.VMEM((TAIL,), jnp.int32),
            pltpu.VMEM((2, CH) + rsa, dta), pltpu.VMEM((2, CH) + rsb, dtb),
            pltpu.VMEM((TAIL,) + rsa, dta), pltpu.VMEM((TAIL,) + rsb, dtb),
            pltpu.SemaphoreType.DMA, pltpu.SemaphoreType.DMA,
            pltpu.SemaphoreType.DMA, pltpu.SemaphoreType.DMA,
            pltpu.SemaphoreType.DMA, pltpu.SemaphoreType.DMA,
        ),
    )
    def gk(tabA, tabB, idxA, idxB, outA, outB,
           ia, ib, ta, tb, ra, rb, tra, trb,
           sga, sgb, swa0, swb0, swa1, swb1):
        wid = lax.axis_index("s") * NC + lax.axis_index("c")
        base = wid * PER_W
        sw = ((swa0, swb0), (swa1, swb1))

        def pair(g, carry):
            @pl.when(g > 0)
            def _():
                for b in range(2):
                    pltpu.make_async_copy(ra.at[b], outA.at[pl.ds(0, CH)],
                                          sw[b][0]).wait()
                    pltpu.make_async_copy(rb.at[b], outB.at[pl.ds(0, CH)],
                                          sw[b][1]).wait()

            copies = []
            for b in range(2):
                off = base + (g * 2 + b) * CH
                pltpu.sync_copy(idxA.at[pl.ds(off, CH)], ia.at[b])
                pltpu.sync_copy(idxB.at[pl.ds(off, CH)], ib.at[b])
                copies.append(
                    (pltpu.async_copy(tabA.at[ia.at[b]], ra.at[b], sga),
                     pltpu.async_copy(tabB.at[ib.at[b]], rb.at[b], sgb)))
            for b in range(2):
                off = base + (g * 2 + b) * CH
                ca, cb = copies[b]
                ca.wait()
                cb.wait()
                pltpu.async_copy(ra.at[b], outA.at[pl.ds(off, CH)], sw[b][0])
                pltpu.async_copy(rb.at[b], outB.at[pl.ds(off, CH)], sw[b][1])
            return carry

        lax.fori_loop(0, NPAIR, pair, 0)
        for b in range(2):
            pltpu.make_async_copy(ra.at[b], outA.at[pl.ds(0, CH)],
                                  sw[b][0]).wait()
            pltpu.make_async_copy(rb.at[b], outB.at[pl.ds(0, CH)],
                                  sw[b][1]).wait()
        off = base + NFULL * CH
        pltpu.sync_copy(idxA.at[pl.ds(off, TAIL)], ta)
        pltpu.sync_copy(idxB.at[pl.ds(off, TAIL)], tb)
        ca = pltpu.async_copy(tabA.at[ta], tra, sga)
        cb = pltpu.async_copy(tabB.at[tb], trb, sgb)
        ca.wait()
        cb.wait()
        pltpu.sync_copy(tra, outA.at[pl.ds(off, TAIL)])
        pltpu.sync_copy(trb, outB.at[pl.ds(off, TAIL)])

    return gk


def _gather_layer(kv_t, q_t, src, dst):
    return _make_gather((HID,), jnp.int32, (HID,), f32)(kv_t, q_t, src, dst)


def _gather_fin(vfin_a, vfin_b, src, dst):
    return _make_gather((2 * HID,), f32, (2 * HID,), f32)(vfin_a, vfin_b,
                                                          src, dst)


@functools.lru_cache(maxsize=None)
def _make_scatter():
    mesh = plsc.VectorSubcoreMesh(core_axis_name="c", subcore_axis_name="s")

    @functools.partial(
        pl.kernel, mesh=mesh,
        out_type=(jax.ShapeDtypeStruct((2 * NPAD, HID), f32),
                  jax.ShapeDtypeStruct((2 * NPAD, HID), f32)),
        scratch_types=(
            pltpu.VMEM((2, CH), jnp.int32),
            pltpu.VMEM((2, CH, HID), f32),
            pltpu.VMEM((TAIL,), jnp.int32),
            pltpu.VMEM((TAIL, HID), f32),
            pltpu.VMEM((ZR, HID), f32),
            pltpu.VMEM_SHARED((NPAD, HID), f32),
            pltpu.SemaphoreType.DMA, pltpu.SemaphoreType.DMA,
        ),
    )
    def sck(contrib_h, arep_h, dst_h, agg_out, den_out,
            idx_v, rows_v, idx_t, rows_t, zb, acc_sh, sl0, sl1):
        c = lax.axis_index("c")
        s = lax.axis_index("s")
        zeros16 = jnp.zeros((16,), f32)

        def zero_zb():
            def zrow(r, carry):
                def zcol(j, cc):
                    zb[r, pl.ds(j * 16, 16)] = zeros16
                    return cc
                return lax.fori_loop(0, HID // 16, zcol, carry)

            lax.fori_loop(0, ZR, zrow, 0)

        zero_zb()

        r0 = s * TROWS
        wid = s * NC + c
        base = wid * PER_W
        sl = (sl0, sl1)

        def zero_acc():
            for b in range(TROWS // ZR):
                pltpu.sync_copy(zb, acc_sh.at[pl.ds(r0 + b * ZR, ZR)])

        def scatter_pass(src_h, out_h):
            def pair(g, carry):
                copies = []
                for b in range(2):
                    off = base + (g * 2 + b) * CH
                    pltpu.sync_copy(dst_h.at[pl.ds(off, CH)], idx_v.at[b])
                    copies.append(pltpu.async_copy(
                        src_h.at[pl.ds(off, CH)], rows_v.at[b], sl[b]))
                for b in range(2):
                    copies[b].wait()
                    pltpu.sync_copy(rows_v.at[b], acc_sh.at[idx_v.at[b]],
                                    add=True)
                return carry

            lax.fori_loop(0, NPAIR, pair, 0)
            off = base + NFULL * CH
            pltpu.sync_copy(dst_h.at[pl.ds(off, TAIL)], idx_t)
            pltpu.sync_copy(src_h.at[pl.ds(off, TAIL)], rows_t)
            pltpu.sync_copy(rows_t, acc_sh.at[idx_t], add=True)
            plsc.subcore_barrier()
            for b in range(TROWS // DCH):
                pltpu.sync_copy(acc_sh.at[pl.ds(r0 + b * DCH, DCH)], zb)
                pltpu.sync_copy(zb,
                                out_h.at[pl.ds(c * NPAD + r0 + b * DCH, DCH)])
            plsc.subcore_barrier()

        zero_acc()
        plsc.subcore_barrier()
        scatter_pass(contrib_h, agg_out)
        zero_zb()
        zero_acc()
        plsc.subcore_barrier()
        scatter_pass(arep_h, den_out)

    return sck


def _scatter(contrib, arep, dst):
    return _make_scatter()(contrib, arep, dst)


def _node_embed_body(nf_r, Wv_r, bv_r, Wq_r, bq_r, Wk_r, bk_r, Wvv_r, bvv_r,
                     v_r, q_r, kv_r):
    v = _dot(nf_r[...], Wv_r[...]) + bv_r[...]
    v_r[...] = v
    q_r[...] = _dot(v, Wq_r[...]) + bq_r[...]
    k = _dot(v, Wk_r[...]) + bk_r[...]
    val = _dot(v, Wvv_r[...]) + bvv_r[...]
    kh = lax.convert_element_type(
        lax.bitcast_convert_type(k.astype(jnp.bfloat16), jnp.uint16),
        jnp.uint32)
    vh = lax.convert_element_type(
        lax.bitcast_convert_type(val.astype(jnp.bfloat16), jnp.uint16),
        jnp.uint32)
    kv_r[...] = lax.bitcast_convert_type(kh | (vh << 16), jnp.int32)


def _node_embed(nf_pad, Wv, bv, Wq, bq, Wk, bk, Wvv, bvv):
    return pl.pallas_call(
        _node_embed_body, grid=(N // NB,),
        in_specs=[_rowspec(NB, 40), _fullspec((40, HID)), _fullspec((1, HID)),
                  _fullspec((HID, HID)), _fullspec((1, HID)),
                  _fullspec((HID, HID)), _fullspec((1, HID)),
                  _fullspec((HID, HID)), _fullspec((1, HID))],
        out_specs=[_rowspec(NB, HID), _rowspec(NB, HID), _rowspec(NB, HID)],
        out_shape=[jax.ShapeDtypeStruct((N, HID), f32),
                   jax.ShapeDtypeStruct((N, HID), f32),
                   jax.ShapeDtypeStruct((N, HID), jnp.int32)],
    )(nf_pad, Wv, bv, Wq, bq, Wk, bk, Wvv, bvv)


def _edge_embed_body(ef_r, d_r, We_r, be_r, e_r, env_r):
    e_r[...] = _dot(ef_r[...], We_r[...]) + be_r[...]
    d = d_r[...]
    x01 = jnp.clip(d / CUT, 0.0, 1.0)
    x2 = x01 * x01
    x3 = x2 * x01
    x4 = x2 * x2
    x5 = x4 * x01
    env_r[...] = jnp.where(d < CUT, 1.0 - 6.0 * x5 + 15.0 * x4 - 10.0 * x3, 0.0)


def _edge_embed(ef_pad, dist, We, be):
    return pl.pallas_call(
        _edge_embed_body, grid=(E // EB,),
        in_specs=[_rowspec(EB, 40), _rowspec(EB, 1),
                  _fullspec((40, HID)), _fullspec((1, HID))],
        out_specs=[_rowspec(EB, HID), _rowspec(EB, 1)],
        out_shape=[jax.ShapeDtypeStruct((E, HID), f32),
                   jax.ShapeDtypeStruct((E, 1), f32)],
    )(ef_pad, dist, We, be)


def _edge_layer_body(e_r, ksvs_r, qd_r, env_r,
                     Wee_r, bee_r, Woe_r, boe_r, g1_r, b1_r,
                     Wf1_r, bf1_r, Wf2_r, bf2_r, g2_r, b2_r,
                     e2_r, contrib_r, arep_r):
    e = e_r[...]
    w = lax.bitcast_convert_type(ksvs_r[...], jnp.uint32)
    ks = lax.bitcast_convert_type(
        lax.convert_element_type(w, jnp.uint16), jnp.bfloat16).astype(f32)
    vs = lax.bitcast_convert_type(
        lax.convert_element_type(w >> 16, jnp.uint16),
        jnp.bfloat16).astype(f32)
    qd = qd_r[...]
    ee = _dot(e, Wee_r[...]) + bee_r[...]
    score = ks * qd * (0.25 * ee)
    s = jnp.clip(_dot(score, _sel_hd()), -5.0, 5.0)
    a = jnp.exp(s) * env_r[...]
    arep = _dot(a, _sel_dh())
    contrib_r[...] = arep * vs
    arep_r[...] = arep
    eh = e + _dot(score, Woe_r[...]) + boe_r[...]
    eh = _lnk(eh, g1_r[...], b1_r[...])
    h = jnp.maximum(_dot(eh, Wf1_r[...]) + bf1_r[...], 0.0)
    e2_r[...] = _lnk(eh + _dot(h, Wf2_r[...]) + bf2_r[...], g2_r[...], b2_r[...])


def _edge_layer(e, ksvs, qd, env, Wee, bee, Woe, boe, g1, b1,
                Wf1, bf1, Wf2, bf2, g2, b2):
    return pl.pallas_call(
        _edge_layer_body, grid=(E // EB,),
        in_specs=[_rowspec(EB, HID), _rowspec(EB, HID),
                  _rowspec(EB, HID), _rowspec(EB, 1),
                  _fullspec((HID, HID)), _fullspec((1, HID)),
                  _fullspec((HID, HID)), _fullspec((1, HID)),
                  _fullspec((1, HID)), _fullspec((1, HID)),
                  _fullspec((HID, FFN)), _fullspec((1, FFN)),
                  _fullspec((FFN, HID)), _fullspec((1, HID)),
                  _fullspec((1, HID)), _fullspec((1, HID))],
        out_specs=[_rowspec(EB, HID), _rowspec(EB, HID), _rowspec(EB, HID)],
        out_shape=[jax.ShapeDtypeStruct((E, HID), f32),
                   jax.ShapeDtypeStruct((E, HID), f32),
                   jax.ShapeDtypeStruct((E, HID), f32)],
    )(e, ksvs, qd, env, Wee, bee, Woe, boe, g1, b1, Wf1, bf1, Wf2, bf2, g2, b2)


def _node_common(v_r, aggp_r, denp_r, Wo_r, bo_r, g1_r, b1_r,
                 Wf1_r, bf1_r, Wf2_r, bf2_r, g2_r, b2_r):
    agg = aggp_r[0] + aggp_r[1]
    denr = denp_r[0] + denp_r[1] + 1e-6
    v_att = agg / denr
    vh = v_r[...] + _dot(v_att, Wo_r[...]) + bo_r[...]
    vh = _lnk(vh, g1_r[...], b1_r[...])
    h = jnp.maximum(_dot(vh, Wf1_r[...]) + bf1_r[...], 0.0)
    return _lnk(vh + _dot(h, Wf2_r[...]) + bf2_r[...], g2_r[...], b2_r[...])


def _node_layer_body(v_r, aggp_r, denp_r, Wo_r, bo_r, g1_r, b1_r,
                     Wf1_r, bf1_r, Wf2_r, bf2_r, g2_r, b2_r,
                     Wq_r, bq_r, Wk_r, bk_r, Wvv_r, bvv_r,
                     v2_r, q_r, kv_r):
    v2 = _node_common(v_r, aggp_r, denp_r, Wo_r, bo_r, g1_r, b1_r,
                      Wf1_r, bf1_r, Wf2_r, bf2_r, g2_r, b2_r)
    v2_r[...] = v2
    q_r[...] = _dot(v2, Wq_r[...]) + bq_r[...]
    k = _dot(v2, Wk_r[...]) + bk_r[...]
    val = _dot(v2, Wvv_r[...]) + bvv_r[...]
    kh = lax.convert_element_type(
        lax.bitcast_convert_type(k.astype(jnp.bfloat16), jnp.uint16),
        jnp.uint32)
    vh = lax.convert_element_type(
        lax.bitcast_convert_type(val.astype(jnp.bfloat16), jnp.uint16),
        jnp.uint32)
    kv_r[...] = lax.bitcast_convert_type(kh | (vh << 16), jnp.int32)


def _node_layer(v, aggp, denp, Wo, bo, g1, b1, Wf1, bf1, Wf2, bf2, g2, b2,
                Wq, bq, Wk, bk, Wvv, bvv):
    wspecs = [_fullspec((HID, HID)), _fullspec((1, HID)),
              _fullspec((1, HID)), _fullspec((1, HID)),
              _fullspec((HID, FFN)), _fullspec((1, FFN)),
              _fullspec((FFN, HID)), _fullspec((1, HID)),
              _fullspec((1, HID)), _fullspec((1, HID))]
    qkvspecs = [_fullspec((HID, HID)), _fullspec((1, HID)),
                _fullspec((HID, HID)), _fullspec((1, HID)),
                _fullspec((HID, HID)), _fullspec((1, HID))]
    return pl.pallas_call(
        _node_layer_body, grid=(N // NB,),
        in_specs=[_rowspec(NB, HID),
                  pl.BlockSpec((2, NB, HID), lambda i: (0, i, 0)),
                  pl.BlockSpec((2, NB, HID), lambda i: (0, i, 0))]
                 + wspecs + qkvspecs,
        out_specs=[_rowspec(NB, HID), _rowspec(NB, HID), _rowspec(NB, HID)],
        out_shape=[jax.ShapeDtypeStruct((N, HID), f32),
                   jax.ShapeDtypeStruct((N, HID), f32),
                   jax.ShapeDtypeStruct((N, HID), jnp.int32)],
    )(v, aggp, denp, Wo, bo, g1, b1, Wf1, bf1, Wf2, bf2, g2, b2,
      Wq, bq, Wk, bk, Wvv, bvv)


def _node_final_body(v_r, aggp_r, denp_r, nf0_r, rad_r,
                     Wo_r, bo_r, g1_r, b1_r, Wf1_r, bf1_r, Wf2_r, bf2_r,
                     g2_r, b2_r, vfin_r, z_r):
    v2 = _node_common(v_r, aggp_r, denp_r, Wo_r, bo_r, g1_r, b1_r,
                      Wf1_r, bf1_r, Wf2_r, bf2_r, g2_r, b2_r)
    vsc = v2 * nf0_r[...]
    vfin_r[:, :HID] = vsc
    lane = lax.broadcasted_iota(jnp.int32, (1, HID), 1)
    vfin_r[:, HID:] = jnp.where(lane == 0, rad_r[...], 0.0)

    @pl.when(pl.program_id(0) == 0)
    def _():
        z_r[...] = jnp.zeros_like(z_r)

    z_r[...] += jnp.sum(vsc, axis=0, keepdims=True)


def _node_final(v, aggp, denp, nf0, rad, Wo, bo, g1, b1, Wf1, bf1, Wf2, bf2,
                g2, b2):
    wspecs = [_fullspec((HID, HID)), _fullspec((1, HID)),
              _fullspec((1, HID)), _fullspec((1, HID)),
              _fullspec((HID, FFN)), _fullspec((1, FFN)),
              _fullspec((FFN, HID)), _fullspec((1, HID)),
              _fullspec((1, HID)), _fullspec((1, HID))]
    return pl.pallas_call(
        _node_final_body, grid=(N // NB,),
        in_specs=[_rowspec(NB, HID),
                  pl.BlockSpec((2, NB, HID), lambda i: (0, i, 0)),
                  pl.BlockSpec((2, NB, HID), lambda i: (0, i, 0)),
                  _rowspec(NB, 1), _rowspec(NB, 1)] + wspecs,
        out_specs=[_rowspec(NB, 2 * HID), _fullspec((1, HID))],
        out_shape=[jax.ShapeDtypeStruct((N, 2 * HID), f32),
                   jax.ShapeDtypeStruct((1, HID), f32)],
    )(v, aggp, denp, nf0, rad, Wo, bo, g1, b1, Wf1, bf1, Wf2, bf2, g2, b2)


def _sigmoid(x):
    return 1.0 / (1.0 + jnp.exp(-x))


def _readout_body(xs_r, xd_r, d_r, it_r, ef0_r,
                  W21_r, b21_r, W22_r, b22_r, W31_r, b31_r, W32_r, b32_r,
                  y_r):
    x = xs_r[:, :HID] + xd_r[:, :HID]
    rsum = xs_r[:, HID:HID + 1] + xd_r[:, HID:HID + 1]
    dist = d_r[...]
    d = dist - rsum
    t = d * 1.25
    V0 = -0.045 * jnp.exp(-(t * t))
    V1 = 0.8 * jnp.where(d < 0, d * d, 0.0)
    V2 = -0.035 * (jnp.where((d > 0) & (d < 2.5), -0.4 * (d - 2.5), 0.0)
                   + jnp.where(d <= 0, 1.0, 0.0))
    V3 = -0.6 * (jnp.where((d > -0.6) & (d < 0), (-5.0 / 3.0) * d, 0.0)
                 + jnp.where(d <= -0.6, 1.0, 0.0))
    mask = jnp.where(dist < CUT, 1.0, 0.0) * ef0_r[...]
    w2 = _sigmoid(_dot(jnp.maximum(_dot(x, W21_r[...]) + b21_r[...], 0.0),
                       W22_r[...]) + b22_r[...]) + 0.5
    w3 = _sigmoid(_dot(jnp.maximum(_dot(x, W31_r[...]) + b31_r[...], 0.0),
                       W32_r[...]) + b32_r[...]) + 0.5
    it1 = it_r[:, 1:2]
    it2 = it_r[:, 2:3]
    t2 = (w2[:, 0:1] * V0 + w2[:, 1:2] * V1 + it1 * w2[:, 2:3] * V2
          + it2 * w2[:, 3:4] * V3)
    t3 = (w3[:, 0:1] * V0 + w3[:, 1:2] * V1 + it1 * w3[:, 2:3] * V2
          + it2 * w3[:, 3:4] * V3)
    p2 = jnp.sum(mask * t2) * 0.5
    p3 = jnp.sum(mask * t3) * 0.5

    @pl.when(pl.program_id(0) == 0)
    def _():
        y_r[...] = jnp.zeros_like(y_r)

    lane = lax.broadcasted_iota(jnp.int32, (1, HID), 1)
    y_r[...] += jnp.where(lane == 0, p2, 0.0) + jnp.where(lane == 1, p3, 0.0)


def _readout(xs, xd, dist, itp, ef0, W21, b21, W22, b22, W31, b31, W32, b32):
    return pl.pallas_call(
        _readout_body, grid=(E // EB,),
        in_specs=[_rowspec(EB, 2 * HID), _rowspec(EB, 2 * HID), _rowspec(EB, 1),
                  _rowspec(EB, 8), _rowspec(EB, 1),
                  _fullspec((HID, HID)), _fullspec((1, HID)),
                  _fullspec((HID, 8)), _fullspec((1, 8)),
                  _fullspec((HID, HID)), _fullspec((1, HID)),
                  _fullspec((HID, 8)), _fullspec((1, 8))],
        out_specs=_fullspec((1, HID)),
        out_shape=jax.ShapeDtypeStruct((1, HID), f32),
    )(xs, xd, dist, itp, ef0, W21, b21, W22, b22, W31, b31, W32, b32)


def _head_body(z_r, y_r, nrot_r, W11_r, b11_r, W12_r, b12_r,
               W41_r, b41_r, W42_r, b42_r, out_r):
    z = z_r[...]
    y1 = (_dot(jnp.maximum(_dot(z, W11_r[...]) + b11_r[...], 0.0),
               W12_r[...]) + b12_r[...])[0, 0]
    w4 = _sigmoid((_dot(jnp.maximum(_dot(z, W41_r[...]) + b41_r[...], 0.0),
                        W42_r[...]) + b42_r[...])[0, 0]) + 0.5
    y2 = y_r[0, 0]
    y3 = y_r[0, 1] / (1.0 + w4 * 0.05846 * nrot_r[0, 0])
    lane = lax.broadcasted_iota(jnp.int32, (1, HID), 1)
    out_r[...] = (jnp.where(lane == 0, y1, 0.0)
                  + jnp.where(lane == 1, y2, 0.0)
                  + jnp.where(lane == 2, y3, 0.0))


def _head(z, y, nrot, W11, b11, W12, b12, W41, b41, W42, b42):
    return pl.pallas_call(
        _head_body,
        out_shape=jax.ShapeDtypeStruct((1, HID), f32),
    )(z, y, nrot, W11, b11, W12, b12, W41, b41, W42, b42)


def kernel(node_feature, edge_feature, vdw_radii, distance, interaction_type,
           edge_index, n_rot, params):
    p = params
    src = edge_index[0].astype(jnp.int32)
    dst = edge_index[1].astype(jnp.int32)
    nf_pad = jnp.pad(node_feature, ((0, 0), (0, 1)))
    ef_pad = jnp.pad(edge_feature, ((0, 0), (0, 1)))
    Wv = jnp.pad(p['Wv_emb'], ((0, 1), (0, 0)))
    We = jnp.pad(p['We_emb'], ((0, 1), (0, 0)))
    dist = distance[:, None]
    itp = jnp.pad(interaction_type, ((0, 0), (0, 5)))
    ef0 = edge_feature[:, 0:1]
    nf0 = node_feature[:, 0:1]
    rad = vdw_radii[:, None]
    nrot = jnp.asarray(n_rot, f32).reshape(1, 1)

    def row(x):
        return x.reshape(1, -1)

    v, q_t, kv_t = _node_embed(nf_pad, Wv, row(p['bv_emb']),
                               p['Wq'][0], row(p['bq'][0]),
                               p['Wk'][0], row(p['bk'][0]),
                               p['Wvv'][0], row(p['bvv'][0]))
    e, env = _edge_embed(ef_pad, dist, We, row(p['be_emb']))

    vfin = None
    z = None
    for l in range(NLAYERS):
        ksvs, qd = _gather_layer(kv_t, q_t, src, dst)
        e, contrib, arep = _edge_layer(
            e, ksvs, qd, env,
            p['Wee'][l], row(p['bee'][l]), p['Woe'][l], row(p['boe'][l]),
            row(p['g1e'][l]), row(p['b1e'][l]),
            p['Wf1e'][l], row(p['bf1e'][l]), p['Wf2e'][l], row(p['bf2e'][l]),
            row(p['g2e'][l]), row(p['b2e'][l]))
        aggf, denf = _scatter(contrib, arep, dst)
        aggp = aggf.reshape(2, NPAD, HID)
        denp = denf.reshape(2, NPAD, HID)
        nw = (p['Wo'][l], row(p['bo'][l]), row(p['g1v'][l]), row(p['b1v'][l]),
              p['Wf1v'][l], row(p['bf1v'][l]), p['Wf2v'][l], row(p['bf2v'][l]),
              row(p['g2v'][l]), row(p['b2v'][l]))
        if l < NLAYERS - 1:
            v, q_t, kv_t = _node_layer(v, aggp, denp, *nw,
                                       p['Wq'][l + 1], row(p['bq'][l + 1]),
                                       p['Wk'][l + 1], row(p['bk'][l + 1]),
                                       p['Wvv'][l + 1], row(p['bvv'][l + 1]))
        else:
            vfin, z = _node_final(v, aggp, denp, nf0, rad, *nw)

    xs, xd = _gather_fin(vfin, vfin, src, dst)
    y23 = _readout(xs, xd, dist, itp, ef0,
                   p['r2_W1'], row(p['r2_b1']),
                   jnp.pad(p['r2_W2'], ((0, 0), (0, 4))),
                   row(jnp.pad(p['r2_b2'], (0, 4))),
                   p['r3_W1'], row(p['r3_b1']),
                   jnp.pad(p['r3_W2'], ((0, 0), (0, 4))),
                   row(jnp.pad(p['r3_b2'], (0, 4))))
    out = _head(z, y23, nrot,
                p['r1_W1'], row(p['r1_b1']),
                jnp.pad(p['r1_W2'], ((0, 0), (0, 7))),
                row(jnp.pad(p['r1_b2'], (0, 7))),
                p['r4_W1'], row(p['r4_b1']),
                jnp.pad(p['r4_W2'], ((0, 0), (0, 7))),
                row(jnp.pad(p['r4_b2'], (0, 7))))
    return out[0, :3]


# speedup vs baseline: 34.9692x; 1.0215x over previous
import functools

import jax
import jax.numpy as jnp
from jax import lax
from jax.experimental import pallas as pl
from jax.experimental.pallas import tpu as pltpu
from jax.experimental.pallas import tpu_sc as plsc

N = 10000
E = 160000
HID = 128
HEADS = 8
DH = HID // HEADS
NLAYERS = 10
FFN = 2 * HID
CUT = 6.5
f32 = jnp.float32

NC = 2
NS = 16
NW = NC * NS
PER_W = E // NW
CH = 104
NFULL = PER_W // CH
NPAIR = NFULL // 2
TAIL = PER_W - NFULL * CH
DCH = 128
NPAD = 10240
TROWS = NPAD // NS
ZR = 128

EB = 2000
NB = 1000

_PREC = jax.lax.Precision.DEFAULT


def _dot(a, b):
    return jnp.dot(a, b, preferred_element_type=f32, precision=_PREC)


def _lnk(x, g, b):
    m = jnp.mean(x, axis=-1, keepdims=True)
    v = jnp.mean((x - m) * (x - m), axis=-1, keepdims=True)
    return (x - m) * lax.rsqrt(v + 1e-5) * g + b


def _sel_hd():
    r = lax.broadcasted_iota(jnp.int32, (HID, HEADS), 0) // DH
    c = lax.broadcasted_iota(jnp.int32, (HID, HEADS), 1)
    return (r == c).astype(f32)


def _sel_dh():
    r = lax.broadcasted_iota(jnp.int32, (HEADS, HID), 0)
    c = lax.broadcasted_iota(jnp.int32, (HEADS, HID), 1) // DH
    return (r == c).astype(f32)


def _rowspec(b, w):
    return pl.BlockSpec((b, w), lambda i: (i, 0))


def _fullspec(shape):
    return pl.BlockSpec(shape, lambda i: tuple(0 for _ in shape))


@functools.lru_cache(maxsize=None)
def _make_gather(rsa, dta, rsb, dtb):
    mesh = plsc.VectorSubcoreMesh(core_axis_name="c", subcore_axis_name="s")

    @functools.partial(
        pl.kernel, mesh=mesh,
        out_type=(jax.ShapeDtypeStruct((E,) + rsa, dta),
                  jax.ShapeDtypeStruct((E,) + rsb, dtb)),
        scratch_types=(
            pltpu.VMEM((2, CH), jnp.int32), pltpu.VMEM((2, CH), jnp.int32),
            pltpu.VMEM((TAIL,), jnp.int32), pltpu.VMEM((TAIL,), jnp.int32),
            pltpu.VMEM((2, CH) + rsa, dta), pltpu.VMEM((2, CH) + rsb, dtb),
            pltpu.VMEM((TAIL,) + rsa, dta), pltpu.VMEM((TAIL,) + rsb, dtb),
            pltpu.SemaphoreType.DMA, pltpu.SemaphoreType.DMA,
            pltpu.SemaphoreType.DMA, pltpu.SemaphoreType.DMA,
            pltpu.SemaphoreType.DMA, pltpu.SemaphoreType.DMA,
        ),
    )
    def gk(tabA, tabB, idxA, idxB, outA, outB,
           ia, ib, ta, tb, ra, rb, tra, trb,
           sga, sgb, swa0, swb0, swa1, swb1):
        wid = lax.axis_index("s") * NC + lax.axis_index("c")
        base = wid * PER_W
        sw = ((swa0, swb0), (swa1, swb1))

        def pair(g, carry):
            @pl.when(g > 0)
            def _():
                for b in range(2):
                    pltpu.make_async_copy(ra.at[b], outA.at[pl.ds(0, CH)],
                                          sw[b][0]).wait()
                    pltpu.make_async_copy(rb.at[b], outB.at[pl.ds(0, CH)],
                                          sw[b][1]).wait()

            copies = []
            for b in range(2):
                off = base + (g * 2 + b) * CH
                pltpu.sync_copy(idxA.at[pl.ds(off, CH)], ia.at[b])
                pltpu.sync_copy(idxB.at[pl.ds(off, CH)], ib.at[b])
                copies.append(
                    (pltpu.async_copy(tabA.at[ia.at[b]], ra.at[b], sga),
                     pltpu.async_copy(tabB.at[ib.at[b]], rb.at[b], sgb)))
            for b in range(2):
                off = base + (g * 2 + b) * CH
                ca, cb = copies[b]
                ca.wait()
                cb.wait()
                pltpu.async_copy(ra.at[b], outA.at[pl.ds(off, CH)], sw[b][0])
                pltpu.async_copy(rb.at[b], outB.at[pl.ds(off, CH)], sw[b][1])
            return carry

        lax.fori_loop(0, NPAIR, pair, 0)
        for b in range(2):
            pltpu.make_async_copy(ra.at[b], outA.at[pl.ds(0, CH)],
                                  sw[b][0]).wait()
            pltpu.make_async_copy(rb.at[b], outB.at[pl.ds(0, CH)],
                                  sw[b][1]).wait()
        off = base + NFULL * CH
        pltpu.sync_copy(idxA.at[pl.ds(off, TAIL)], ta)
        pltpu.sync_copy(idxB.at[pl.ds(off, TAIL)], tb)
        ca = pltpu.async_copy(tabA.at[ta], tra, sga)
        cb = pltpu.async_copy(tabB.at[tb], trb, sgb)
        ca.wait()
        cb.wait()
        pltpu.sync_copy(tra, outA.at[pl.ds(off, TAIL)])
        pltpu.sync_copy(trb, outB.at[pl.ds(off, TAIL)])

    return gk


def _gather_layer(kv_t, q_t, src, dst):
    return _make_gather((HID,), jnp.int32, (HID,), f32)(kv_t, q_t, src, dst)


def _gather_fin(vfin_a, vfin_b, src, dst):
    return _make_gather((2 * HID,), f32, (2 * HID,), f32)(vfin_a, vfin_b,
                                                          src, dst)


@functools.lru_cache(maxsize=None)
def _make_scatter():
    mesh = plsc.VectorSubcoreMesh(core_axis_name="c", subcore_axis_name="s")

    @functools.partial(
        pl.kernel, mesh=mesh,
        out_type=(jax.ShapeDtypeStruct((2 * NPAD, HID), f32),
                  jax.ShapeDtypeStruct((2 * NPAD, HID), f32)),
        scratch_types=(
            pltpu.VMEM((2, CH), jnp.int32),
            pltpu.VMEM((2, CH, HID), f32),
            pltpu.VMEM((TAIL,), jnp.int32),
            pltpu.VMEM((TAIL, HID), f32),
            pltpu.VMEM((ZR, HID), f32),
            pltpu.VMEM_SHARED((NPAD, HID), f32),
            pltpu.SemaphoreType.DMA, pltpu.SemaphoreType.DMA,
        ),
    )
    def sck(contrib_h, arep_h, dst_h, agg_out, den_out,
            idx_v, rows_v, idx_t, rows_t, zb, acc_sh, sl0, sl1):
        c = lax.axis_index("c")
        s = lax.axis_index("s")
        zeros16 = jnp.zeros((16,), f32)

        def zero_zb():
            def zrow(r, carry):
                def zcol(j, cc):
                    zb[r, pl.ds(j * 16, 16)] = zeros16
                    return cc
                return lax.fori_loop(0, HID // 16, zcol, carry)

            lax.fori_loop(0, ZR, zrow, 0)

        zero_zb()

        r0 = s * TROWS
        wid = s * NC + c
        base = wid * PER_W
        sl = (sl0, sl1)

        def zero_acc():
            for b in range(TROWS // ZR):
                pltpu.sync_copy(zb, acc_sh.at[pl.ds(r0 + b * ZR, ZR)])

        def scatter_pass(src_h, out_h):
            def pair(g, carry):
                copies = []
                for b in range(2):
                    off = base + (g * 2 + b) * CH
                    pltpu.sync_copy(dst_h.at[pl.ds(off, CH)], idx_v.at[b])
                    copies.append(pltpu.async_copy(
                        src_h.at[pl.ds(off, CH)], rows_v.at[b], sl[b]))
                for b in range(2):
                    copies[b].wait()
                    pltpu.sync_copy(rows_v.at[b], acc_sh.at[idx_v.at[b]],
                                    add=True)
                return carry

            lax.fori_loop(0, NPAIR, pair, 0)
            off = base + NFULL * CH
            pltpu.sync_copy(dst_h.at[pl.ds(off, TAIL)], idx_t)
            pltpu.sync_copy(src_h.at[pl.ds(off, TAIL)], rows_t)
            pltpu.sync_copy(rows_t, acc_sh.at[idx_t], add=True)
            plsc.subcore_barrier()
            for b in range(TROWS // DCH):
                pltpu.sync_copy(acc_sh.at[pl.ds(r0 + b * DCH, DCH)], zb)
                pltpu.sync_copy(zb,
                                out_h.at[pl.ds(c * NPAD + r0 + b * DCH, DCH)])
            plsc.subcore_barrier()

        zero_acc()
        plsc.subcore_barrier()
        scatter_pass(contrib_h, agg_out)
        zero_zb()
        zero_acc()
        plsc.subcore_barrier()
        scatter_pass(arep_h, den_out)

    return sck


def _scatter(contrib, arep, dst):
    return _make_scatter()(contrib, arep, dst)


def _node_embed_body(nf_r, Wv_r, bv_r, Wq_r, bq_r, Wk_r, bk_r, Wvv_r, bvv_r,
                     v_r, q_r, kv_r):
    v = _dot(nf_r[...], Wv_r[...]) + bv_r[...]
    v_r[...] = v
    q_r[...] = _dot(v, Wq_r[...]) + bq_r[...]
    k = _dot(v, Wk_r[...]) + bk_r[...]
    val = _dot(v, Wvv_r[...]) + bvv_r[...]
    kh = lax.convert_element_type(
        lax.bitcast_convert_type(k.astype(jnp.bfloat16), jnp.uint16),
        jnp.uint32)
    vh = lax.convert_element_type(
        lax.bitcast_convert_type(val.astype(jnp.bfloat16), jnp.uint16),
        jnp.uint32)
    kv_r[...] = lax.bitcast_convert_type(kh | (vh << 16), jnp.int32)


def _node_embed(nf_pad, Wv, bv, Wq, bq, Wk, bk, Wvv, bvv):
    return pl.pallas_call(
        _node_embed_body, grid=(N // NB,),
        in_specs=[_rowspec(NB, 40), _fullspec((40, HID)), _fullspec((1, HID)),
                  _fullspec((HID, HID)), _fullspec((1, HID)),
                  _fullspec((HID, HID)), _fullspec((1, HID)),
                  _fullspec((HID, HID)), _fullspec((1, HID))],
        out_specs=[_rowspec(NB, HID), _rowspec(NB, HID), _rowspec(NB, HID)],
        out_shape=[jax.ShapeDtypeStruct((N, HID), f32),
                   jax.ShapeDtypeStruct((N, HID), f32),
                   jax.ShapeDtypeStruct((N, HID), jnp.int32)],
    )(nf_pad, Wv, bv, Wq, bq, Wk, bk, Wvv, bvv)


def _edge_embed_body(ef_r, d_r, We_r, be_r, e_r, env_r):
    e_r[...] = (_dot(ef_r[...], We_r[...]) + be_r[...]).astype(jnp.bfloat16)
    d = d_r[...]
    x01 = jnp.clip(d / CUT, 0.0, 1.0)
    x2 = x01 * x01
    x3 = x2 * x01
    x4 = x2 * x2
    x5 = x4 * x01
    env_r[...] = jnp.where(d < CUT, 1.0 - 6.0 * x5 + 15.0 * x4 - 10.0 * x3, 0.0)


def _edge_embed(ef_pad, dist, We, be):
    return pl.pallas_call(
        _edge_embed_body, grid=(E // EB,),
        in_specs=[_rowspec(EB, 40), _rowspec(EB, 1),
                  _fullspec((40, HID)), _fullspec((1, HID))],
        out_specs=[_rowspec(EB, HID), _rowspec(EB, 1)],
        out_shape=[jax.ShapeDtypeStruct((E, HID), jnp.bfloat16),
                   jax.ShapeDtypeStruct((E, 1), f32)],
    )(ef_pad, dist, We, be)


def _edge_layer_body(e_r, ksvs_r, qd_r, env_r,
                     Wee_r, bee_r, Woe_r, boe_r, g1_r, b1_r,
                     Wf1_r, bf1_r, Wf2_r, bf2_r, g2_r, b2_r,
                     e2_r, contrib_r, arep_r):
    e = e_r[...].astype(f32)
    w = lax.bitcast_convert_type(ksvs_r[...], jnp.uint32)
    ks = lax.bitcast_convert_type(
        lax.convert_element_type(w, jnp.uint16), jnp.bfloat16).astype(f32)
    vs = lax.bitcast_convert_type(
        lax.convert_element_type(w >> 16, jnp.uint16),
        jnp.bfloat16).astype(f32)
    qd = qd_r[...]
    ee = _dot(e, Wee_r[...]) + bee_r[...]
    score = ks * qd * (0.25 * ee)
    s = jnp.clip(_dot(score, _sel_hd()), -5.0, 5.0)
    a = jnp.exp(s) * env_r[...]
    arep = _dot(a, _sel_dh())
    contrib_r[...] = arep * vs
    arep_r[...] = arep
    eh = e + _dot(score, Woe_r[...]) + boe_r[...]
    eh = _lnk(eh, g1_r[...], b1_r[...])
    h = jnp.maximum(_dot(eh, Wf1_r[...]) + bf1_r[...], 0.0)
    e2_r[...] = _lnk(eh + _dot(h, Wf2_r[...]) + bf2_r[...], g2_r[...],
                     b2_r[...]).astype(jnp.bfloat16)


def _edge_layer(e, ksvs, qd, env, Wee, bee, Woe, boe, g1, b1,
                Wf1, bf1, Wf2, bf2, g2, b2):
    return pl.pallas_call(
        _edge_layer_body, grid=(E // EB,),
        in_specs=[_rowspec(EB, HID), _rowspec(EB, HID),
                  _rowspec(EB, HID), _rowspec(EB, 1),
                  _fullspec((HID, HID)), _fullspec((1, HID)),
                  _fullspec((HID, HID)), _fullspec((1, HID)),
                  _fullspec((1, HID)), _fullspec((1, HID)),
                  _fullspec((HID, FFN)), _fullspec((1, FFN)),
                  _fullspec((FFN, HID)), _fullspec((1, HID)),
                  _fullspec((1, HID)), _fullspec((1, HID))],
        out_specs=[_rowspec(EB, HID), _rowspec(EB, HID), _rowspec(EB, HID)],
        out_shape=[jax.ShapeDtypeStruct((E, HID), jnp.bfloat16),
                   jax.ShapeDtypeStruct((E, HID), f32),
                   jax.ShapeDtypeStruct((E, HID), f32)],
    )(e, ksvs, qd, env, Wee, bee, Woe, boe, g1, b1, Wf1, bf1, Wf2, bf2, g2, b2)


def _node_common(v_r, aggp_r, denp_r, Wo_r, bo_r, g1_r, b1_r,
                 Wf1_r, bf1_r, Wf2_r, bf2_r, g2_r, b2_r):
    agg = aggp_r[0] + aggp_r[1]
    denr = denp_r[0] + denp_r[1] + 1e-6
    v_att = agg / denr
    vh = v_r[...] + _dot(v_att, Wo_r[...]) + bo_r[...]
    vh = _lnk(vh, g1_r[...], b1_r[...])
    h = jnp.maximum(_dot(vh, Wf1_r[...]) + bf1_r[...], 0.0)
    return _lnk(vh + _dot(h, Wf2_r[...]) + bf2_r[...], g2_r[...], b2_r[...])


def _node_layer_body(v_r, aggp_r, denp_r, Wo_r, bo_r, g1_r, b1_r,
                     Wf1_r, bf1_r, Wf2_r, bf2_r, g2_r, b2_r,
                     Wq_r, bq_r, Wk_r, bk_r, Wvv_r, bvv_r,
                     v2_r, q_r, kv_r):
    v2 = _node_common(v_r, aggp_r, denp_r, Wo_r, bo_r, g1_r, b1_r,
                      Wf1_r, bf1_r, Wf2_r, bf2_r, g2_r, b2_r)
    v2_r[...] = v2
    q_r[...] = _dot(v2, Wq_r[...]) + bq_r[...]
    k = _dot(v2, Wk_r[...]) + bk_r[...]
    val = _dot(v2, Wvv_r[...]) + bvv_r[...]
    kh = lax.convert_element_type(
        lax.bitcast_convert_type(k.astype(jnp.bfloat16), jnp.uint16),
        jnp.uint32)
    vh = lax.convert_element_type(
        lax.bitcast_convert_type(val.astype(jnp.bfloat16), jnp.uint16),
        jnp.uint32)
    kv_r[...] = lax.bitcast_convert_type(kh | (vh << 16), jnp.int32)


def _node_layer(v, aggp, denp, Wo, bo, g1, b1, Wf1, bf1, Wf2, bf2, g2, b2,
                Wq, bq, Wk, bk, Wvv, bvv):
    wspecs = [_fullspec((HID, HID)), _fullspec((1, HID)),
              _fullspec((1, HID)), _fullspec((1, HID)),
              _fullspec((HID, FFN)), _fullspec((1, FFN)),
              _fullspec((FFN, HID)), _fullspec((1, HID)),
              _fullspec((1, HID)), _fullspec((1, HID))]
    qkvspecs = [_fullspec((HID, HID)), _fullspec((1, HID)),
                _fullspec((HID, HID)), _fullspec((1, HID)),
                _fullspec((HID, HID)), _fullspec((1, HID))]
    return pl.pallas_call(
        _node_layer_body, grid=(N // NB,),
        in_specs=[_rowspec(NB, HID),
                  pl.BlockSpec((2, NB, HID), lambda i: (0, i, 0)),
                  pl.BlockSpec((2, NB, HID), lambda i: (0, i, 0))]
                 + wspecs + qkvspecs,
        out_specs=[_rowspec(NB, HID), _rowspec(NB, HID), _rowspec(NB, HID)],
        out_shape=[jax.ShapeDtypeStruct((N, HID), f32),
                   jax.ShapeDtypeStruct((N, HID), f32),
                   jax.ShapeDtypeStruct((N, HID), jnp.int32)],
    )(v, aggp, denp, Wo, bo, g1, b1, Wf1, bf1, Wf2, bf2, g2, b2,
      Wq, bq, Wk, bk, Wvv, bvv)


def _node_final_body(v_r, aggp_r, denp_r, nf0_r, rad_r,
                     Wo_r, bo_r, g1_r, b1_r, Wf1_r, bf1_r, Wf2_r, bf2_r,
                     g2_r, b2_r, vfin_r, z_r):
    v2 = _node_common(v_r, aggp_r, denp_r, Wo_r, bo_r, g1_r, b1_r,
                      Wf1_r, bf1_r, Wf2_r, bf2_r, g2_r, b2_r)
    vsc = v2 * nf0_r[...]
    vfin_r[:, :HID] = vsc
    lane = lax.broadcasted_iota(jnp.int32, (1, HID), 1)
    vfin_r[:, HID:] = jnp.where(lane == 0, rad_r[...], 0.0)

    @pl.when(pl.program_id(0) == 0)
    def _():
        z_r[...] = jnp.zeros_like(z_r)

    z_r[...] += jnp.sum(vsc, axis=0, keepdims=True)


def _node_final(v, aggp, denp, nf0, rad, Wo, bo, g1, b1, Wf1, bf1, Wf2, bf2,
                g2, b2):
    wspecs = [_fullspec((HID, HID)), _fullspec((1, HID)),
              _fullspec((1, HID)), _fullspec((1, HID)),
              _fullspec((HID, FFN)), _fullspec((1, FFN)),
              _fullspec((FFN, HID)), _fullspec((1, HID)),
              _fullspec((1, HID)), _fullspec((1, HID))]
    return pl.pallas_call(
        _node_final_body, grid=(N // NB,),
        in_specs=[_rowspec(NB, HID),
                  pl.BlockSpec((2, NB, HID), lambda i: (0, i, 0)),
                  pl.BlockSpec((2, NB, HID), lambda i: (0, i, 0)),
                  _rowspec(NB, 1), _rowspec(NB, 1)] + wspecs,
        out_specs=[_rowspec(NB, 2 * HID), _fullspec((1, HID))],
        out_shape=[jax.ShapeDtypeStruct((N, 2 * HID), f32),
                   jax.ShapeDtypeStruct((1, HID), f32)],
    )(v, aggp, denp, nf0, rad, Wo, bo, g1, b1, Wf1, bf1, Wf2, bf2, g2, b2)


def _sigmoid(x):
    return 1.0 / (1.0 + jnp.exp(-x))


def _readout_body(xs_r, xd_r, d_r, it_r, ef0_r,
                  W21_r, b21_r, W22_r, b22_r, W31_r, b31_r, W32_r, b32_r,
                  y_r):
    x = xs_r[:, :HID] + xd_r[:, :HID]
    rsum = xs_r[:, HID:HID + 1] + xd_r[:, HID:HID + 1]
    dist = d_r[...]
    d = dist - rsum
    t = d * 1.25
    V0 = -0.045 * jnp.exp(-(t * t))
    V1 = 0.8 * jnp.where(d < 0, d * d, 0.0)
    V2 = -0.035 * (jnp.where((d > 0) & (d < 2.5), -0.4 * (d - 2.5), 0.0)
                   + jnp.where(d <= 0, 1.0, 0.0))
    V3 = -0.6 * (jnp.where((d > -0.6) & (d < 0), (-5.0 / 3.0) * d, 0.0)
                 + jnp.where(d <= -0.6, 1.0, 0.0))
    mask = jnp.where(dist < CUT, 1.0, 0.0) * ef0_r[...]
    w2 = _sigmoid(_dot(jnp.maximum(_dot(x, W21_r[...]) + b21_r[...], 0.0),
                       W22_r[...]) + b22_r[...]) + 0.5
    w3 = _sigmoid(_dot(jnp.maximum(_dot(x, W31_r[...]) + b31_r[...], 0.0),
                       W32_r[...]) + b32_r[...]) + 0.5
    it1 = it_r[:, 1:2]
    it2 = it_r[:, 2:3]
    t2 = (w2[:, 0:1] * V0 + w2[:, 1:2] * V1 + it1 * w2[:, 2:3] * V2
          + it2 * w2[:, 3:4] * V3)
    t3 = (w3[:, 0:1] * V0 + w3[:, 1:2] * V1 + it1 * w3[:, 2:3] * V2
          + it2 * w3[:, 3:4] * V3)
    p2 = jnp.sum(mask * t2) * 0.5
    p3 = jnp.sum(mask * t3) * 0.5

    @pl.when(pl.program_id(0) == 0)
    def _():
        y_r[...] = jnp.zeros_like(y_r)

    lane = lax.broadcasted_iota(jnp.int32, (1, HID), 1)
    y_r[...] += jnp.where(lane == 0, p2, 0.0) + jnp.where(lane == 1, p3, 0.0)


def _readout(xs, xd, dist, itp, ef0, W21, b21, W22, b22, W31, b31, W32, b32):
    return pl.pallas_call(
        _readout_body, grid=(E // EB,),
        in_specs=[_rowspec(EB, 2 * HID), _rowspec(EB, 2 * HID), _rowspec(EB, 1),
                  _rowspec(EB, 8), _rowspec(EB, 1),
                  _fullspec((HID, HID)), _fullspec((1, HID)),
                  _fullspec((HID, 8)), _fullspec((1, 8)),
                  _fullspec((HID, HID)), _fullspec((1, HID)),
                  _fullspec((HID, 8)), _fullspec((1, 8))],
        out_specs=_fullspec((1, HID)),
        out_shape=jax.ShapeDtypeStruct((1, HID), f32),
    )(xs, xd, dist, itp, ef0, W21, b21, W22, b22, W31, b31, W32, b32)


def _head_body(z_r, y_r, nrot_r, W11_r, b11_r, W12_r, b12_r,
               W41_r, b41_r, W42_r, b42_r, out_r):
    z = z_r[...]
    y1 = (_dot(jnp.maximum(_dot(z, W11_r[...]) + b11_r[...], 0.0),
               W12_r[...]) + b12_r[...])[0, 0]
    w4 = _sigmoid((_dot(jnp.maximum(_dot(z, W41_r[...]) + b41_r[...], 0.0),
                        W42_r[...]) + b42_r[...])[0, 0]) + 0.5
    y2 = y_r[0, 0]
    y3 = y_r[0, 1] / (1.0 + w4 * 0.05846 * nrot_r[0, 0])
    lane = lax.broadcasted_iota(jnp.int32, (1, HID), 1)
    out_r[...] = (jnp.where(lane == 0, y1, 0.0)
                  + jnp.where(lane == 1, y2, 0.0)
                  + jnp.where(lane == 2, y3, 0.0))


def _head(z, y, nrot, W11, b11, W12, b12, W41, b41, W42, b42):
    return pl.pallas_call(
        _head_body,
        out_shape=jax.ShapeDtypeStruct((1, HID), f32),
    )(z, y, nrot, W11, b11, W12, b12, W41, b41, W42, b42)


def kernel(node_feature, edge_feature, vdw_radii, distance, interaction_type,
           edge_index, n_rot, params):
    p = params
    src = edge_index[0].astype(jnp.int32)
    dst = edge_index[1].astype(jnp.int32)
    nf_pad = jnp.pad(node_feature, ((0, 0), (0, 1)))
    ef_pad = jnp.pad(edge_feature, ((0, 0), (0, 1)))
    Wv = jnp.pad(p['Wv_emb'], ((0, 1), (0, 0)))
    We = jnp.pad(p['We_emb'], ((0, 1), (0, 0)))
    dist = distance[:, None]
    itp = jnp.pad(interaction_type, ((0, 0), (0, 5)))
    ef0 = edge_feature[:, 0:1]
    nf0 = node_feature[:, 0:1]
    rad = vdw_radii[:, None]
    nrot = jnp.asarray(n_rot, f32).reshape(1, 1)

    def row(x):
        return x.reshape(1, -1)

    v, q_t, kv_t = _node_embed(nf_pad, Wv, row(p['bv_emb']),
                               p['Wq'][0], row(p['bq'][0]),
                               p['Wk'][0], row(p['bk'][0]),
                               p['Wvv'][0], row(p['bvv'][0]))
    e, env = _edge_embed(ef_pad, dist, We, row(p['be_emb']))

    vfin = None
    z = None
    for l in range(NLAYERS):
        ksvs, qd = _gather_layer(kv_t, q_t, src, dst)
        e, contrib, arep = _edge_layer(
            e, ksvs, qd, env,
            p['Wee'][l], row(p['bee'][l]), p['Woe'][l], row(p['boe'][l]),
            row(p['g1e'][l]), row(p['b1e'][l]),
            p['Wf1e'][l], row(p['bf1e'][l]), p['Wf2e'][l], row(p['bf2e'][l]),
            row(p['g2e'][l]), row(p['b2e'][l]))
        aggf, denf = _scatter(contrib, arep, dst)
        aggp = aggf.reshape(2, NPAD, HID)
        denp = denf.reshape(2, NPAD, HID)
        nw = (p['Wo'][l], row(p['bo'][l]), row(p['g1v'][l]), row(p['b1v'][l]),
              p['Wf1v'][l], row(p['bf1v'][l]), p['Wf2v'][l], row(p['bf2v'][l]),
              row(p['g2v'][l]), row(p['b2v'][l]))
        if l < NLAYERS - 1:
            v, q_t, kv_t = _node_layer(v, aggp, denp, *nw,
                                       p['Wq'][l + 1], row(p['bq'][l + 1]),
                                       p['Wk'][l + 1], row(p['bk'][l + 1]),
                                       p['Wvv'][l + 1], row(p['bvv'][l + 1]))
        else:
            vfin, z = _node_final(v, aggp, denp, nf0, rad, *nw)

    xs, xd = _gather_fin(vfin, vfin, src, dst)
    y23 = _readout(xs, xd, dist, itp, ef0,
                   p['r2_W1'], row(p['r2_b1']),
                   jnp.pad(p['r2_W2'], ((0, 0), (0, 4))),
                   row(jnp.pad(p['r2_b2'], (0, 4))),
                   p['r3_W1'], row(p['r3_b1']),
                   jnp.pad(p['r3_W2'], ((0, 0), (0, 4))),
                   row(jnp.pad(p['r3_b2'], (0, 4))))
    out = _head(z, y23, nrot,
                p['r1_W1'], row(p['r1_b1']),
                jnp.pad(p['r1_W2'], ((0, 0), (0, 7))),
                row(jnp.pad(p['r1_b2'], (0, 7))),
                p['r4_W1'], row(p['r4_b1']),
                jnp.pad(p['r4_W2'], ((0, 0), (0, 7))),
                row(jnp.pad(p['r4_b2'], (0, 7))))
    return out[0, :3]


# speedup vs baseline: 35.2206x; 1.0072x over previous
import functools

import jax
import jax.numpy as jnp
from jax import lax
from jax.experimental import pallas as pl
from jax.experimental.pallas import tpu as pltpu
from jax.experimental.pallas import tpu_sc as plsc

N = 10000
E = 160000
HID = 128
HEADS = 8
DH = HID // HEADS
NLAYERS = 10
FFN = 2 * HID
CUT = 6.5
f32 = jnp.float32

NC = 2
NS = 16
NW = NC * NS
PER_W = E // NW
CH = 104
NFULL = PER_W // CH
NPAIR = NFULL // 2
TAIL = PER_W - NFULL * CH
DCH = 128
NPAD = 10240
TROWS = NPAD // NS
ZR = 128

EB = 2000
NB = 1000

_PREC = jax.lax.Precision.DEFAULT


def _dot(a, b):
    return jnp.dot(a, b, preferred_element_type=f32, precision=_PREC)


def _lnk(x, g, b):
    m = jnp.mean(x, axis=-1, keepdims=True)
    v = jnp.mean((x - m) * (x - m), axis=-1, keepdims=True)
    return (x - m) * lax.rsqrt(v + 1e-5) * g + b


def _sel_hd():
    r = lax.broadcasted_iota(jnp.int32, (HID, HEADS), 0) // DH
    c = lax.broadcasted_iota(jnp.int32, (HID, HEADS), 1)
    return (r == c).astype(f32)


def _sel_dh():
    r = lax.broadcasted_iota(jnp.int32, (HEADS, HID), 0)
    c = lax.broadcasted_iota(jnp.int32, (HEADS, HID), 1) // DH
    return (r == c).astype(f32)


def _rowspec(b, w):
    return pl.BlockSpec((b, w), lambda i: (i, 0))


def _fullspec(shape):
    return pl.BlockSpec(shape, lambda i: tuple(0 for _ in shape))


@functools.lru_cache(maxsize=None)
def _make_gather(rsa, dta, rsb, dtb):
    mesh = plsc.VectorSubcoreMesh(core_axis_name="c", subcore_axis_name="s")

    @functools.partial(
        pl.kernel, mesh=mesh,
        out_type=(jax.ShapeDtypeStruct((E,) + rsa, dta),
                  jax.ShapeDtypeStruct((E,) + rsb, dtb)),
        scratch_types=(
            pltpu.VMEM((2, CH), jnp.int32), pltpu.VMEM((2, CH), jnp.int32),
            pltpu.VMEM((TAIL,), jnp.int32), pltpu.VMEM((TAIL,), jnp.int32),
            pltpu.VMEM((2, CH) + rsa, dta), pltpu.VMEM((2, CH) + rsb, dtb),
            pltpu.VMEM((TAIL,) + rsa, dta), pltpu.VMEM((TAIL,) + rsb, dtb),
            pltpu.SemaphoreType.DMA, pltpu.SemaphoreType.DMA,
            pltpu.SemaphoreType.DMA, pltpu.SemaphoreType.DMA,
            pltpu.SemaphoreType.DMA, pltpu.SemaphoreType.DMA,
        ),
    )
    def gk(tabA, tabB, idxA, idxB, outA, outB,
           ia, ib, ta, tb, ra, rb, tra, trb,
           sga, sgb, swa0, swb0, swa1, swb1):
        wid = lax.axis_index("s") * NC + lax.axis_index("c")
        base = wid * PER_W
        sw = ((swa0, swb0), (swa1, swb1))

        def pair(g, carry):
            @pl.when(g > 0)
            def _():
                for b in range(2):
                    pltpu.make_async_copy(ra.at[b], outA.at[pl.ds(0, CH)],
                                          sw[b][0]).wait()
                    pltpu.make_async_copy(rb.at[b], outB.at[pl.ds(0, CH)],
                                          sw[b][1]).wait()

            copies = []
            for b in range(2):
                off = base + (g * 2 + b) * CH
                pltpu.sync_copy(idxA.at[pl.ds(off, CH)], ia.at[b])
                pltpu.sync_copy(idxB.at[pl.ds(off, CH)], ib.at[b])
                copies.append(
                    (pltpu.async_copy(tabA.at[ia.at[b]], ra.at[b], sga),
                     pltpu.async_copy(tabB.at[ib.at[b]], rb.at[b], sgb)))
            for b in range(2):
                off = base + (g * 2 + b) * CH
                ca, cb = copies[b]
                ca.wait()
                cb.wait()
                pltpu.async_copy(ra.at[b], outA.at[pl.ds(off, CH)], sw[b][0])
                pltpu.async_copy(rb.at[b], outB.at[pl.ds(off, CH)], sw[b][1])
            return carry

        lax.fori_loop(0, NPAIR, pair, 0)
        for b in range(2):
            pltpu.make_async_copy(ra.at[b], outA.at[pl.ds(0, CH)],
                                  sw[b][0]).wait()
            pltpu.make_async_copy(rb.at[b], outB.at[pl.ds(0, CH)],
                                  sw[b][1]).wait()
        off = base + NFULL * CH
        pltpu.sync_copy(idxA.at[pl.ds(off, TAIL)], ta)
        pltpu.sync_copy(idxB.at[pl.ds(off, TAIL)], tb)
        ca = pltpu.async_copy(tabA.at[ta], tra, sga)
        cb = pltpu.async_copy(tabB.at[tb], trb, sgb)
        ca.wait()
        cb.wait()
        pltpu.sync_copy(tra, outA.at[pl.ds(off, TAIL)])
        pltpu.sync_copy(trb, outB.at[pl.ds(off, TAIL)])

    return gk


def _gather_layer(kv_t, q_t, src, dst):
    return _make_gather((HID,), jnp.int32, (HID,), f32)(kv_t, q_t, src, dst)


def _gather_fin(vfin_a, vfin_b, src, dst):
    return _make_gather((2 * HID,), f32, (2 * HID,), f32)(vfin_a, vfin_b,
                                                          src, dst)


@functools.lru_cache(maxsize=None)
def _make_scatter():
    mesh = plsc.VectorSubcoreMesh(core_axis_name="c", subcore_axis_name="s")

    @functools.partial(
        pl.kernel, mesh=mesh,
        out_type=(jax.ShapeDtypeStruct((2 * NPAD, HID), f32),
                  jax.ShapeDtypeStruct((2 * NPAD, HID), f32)),
        scratch_types=(
            pltpu.VMEM((2, CH), jnp.int32),
            pltpu.VMEM((2, CH, HID), f32),
            pltpu.VMEM((TAIL,), jnp.int32),
            pltpu.VMEM((TAIL, HID), f32),
            pltpu.VMEM((ZR, HID), f32),
            pltpu.VMEM_SHARED((NPAD, HID), f32),
            pltpu.SemaphoreType.DMA, pltpu.SemaphoreType.DMA,
        ),
    )
    def sck(contrib_h, arep_h, dst_h, agg_out, den_out,
            idx_v, rows_v, idx_t, rows_t, zb, acc_sh, sl0, sl1):
        c = lax.axis_index("c")
        s = lax.axis_index("s")
        zeros16 = jnp.zeros((16,), f32)

        def zero_zb():
            def zrow(r, carry):
                def zcol(j, cc):
                    zb[r, pl.ds(j * 16, 16)] = zeros16
                    return cc
                return lax.fori_loop(0, HID // 16, zcol, carry)

            lax.fori_loop(0, ZR, zrow, 0)

        zero_zb()

        r0 = s * TROWS
        wid = s * NC + c
        base = wid * PER_W
        sl = (sl0, sl1)

        def zero_acc():
            for b in range(TROWS // ZR):
                pltpu.sync_copy(zb, acc_sh.at[pl.ds(r0 + b * ZR, ZR)])

        def scatter_pass(src_h, out_h):
            def pair(g, carry):
                copies = []
                for b in range(2):
                    off = base + (g * 2 + b) * CH
                    pltpu.sync_copy(dst_h.at[pl.ds(off, CH)], idx_v.at[b])
                    copies.append(pltpu.async_copy(
                        src_h.at[pl.ds(off, CH)], rows_v.at[b], sl[b]))
                for b in range(2):
                    copies[b].wait()
                    pltpu.sync_copy(rows_v.at[b], acc_sh.at[idx_v.at[b]],
                                    add=True)
                return carry

            lax.fori_loop(0, NPAIR, pair, 0)
            off = base + NFULL * CH
            pltpu.sync_copy(dst_h.at[pl.ds(off, TAIL)], idx_t)
            pltpu.sync_copy(src_h.at[pl.ds(off, TAIL)], rows_t)
            pltpu.sync_copy(rows_t, acc_sh.at[idx_t], add=True)
            plsc.subcore_barrier()
            for b in range(TROWS // DCH):
                pltpu.sync_copy(acc_sh.at[pl.ds(r0 + b * DCH, DCH)], zb)
                pltpu.sync_copy(zb,
                                out_h.at[pl.ds(c * NPAD + r0 + b * DCH, DCH)])
            plsc.subcore_barrier()

        zero_acc()
        plsc.subcore_barrier()
        scatter_pass(contrib_h, agg_out)
        zero_zb()
        zero_acc()
        plsc.subcore_barrier()
        scatter_pass(arep_h, den_out)

    return sck


def _scatter(contrib, arep, dst):
    return _make_scatter()(contrib, arep, dst)


def _node_embed_body(nf_r, Wv_r, bv_r, Wq_r, bq_r, Wk_r, bk_r, Wvv_r, bvv_r,
                     v_r, q_r, kv_r):
    v = _dot(nf_r[...], Wv_r[...]) + bv_r[...]
    v_r[...] = v
    q_r[...] = _dot(v, Wq_r[...]) + bq_r[...]
    k = _dot(v, Wk_r[...]) + bk_r[...]
    val = _dot(v, Wvv_r[...]) + bvv_r[...]
    kh = lax.convert_element_type(
        lax.bitcast_convert_type(k.astype(jnp.bfloat16), jnp.uint16),
        jnp.uint32)
    vh = lax.convert_element_type(
        lax.bitcast_convert_type(val.astype(jnp.bfloat16), jnp.uint16),
        jnp.uint32)
    kv_r[...] = lax.bitcast_convert_type(kh | (vh << 16), jnp.int32)


def _node_embed(nf_pad, Wv, bv, Wq, bq, Wk, bk, Wvv, bvv):
    return pl.pallas_call(
        _node_embed_body, grid=(N // NB,),
        in_specs=[_rowspec(NB, 40), _fullspec((40, HID)), _fullspec((1, HID)),
                  _fullspec((HID, HID)), _fullspec((1, HID)),
                  _fullspec((HID, HID)), _fullspec((1, HID)),
                  _fullspec((HID, HID)), _fullspec((1, HID))],
        out_specs=[_rowspec(NB, HID), _rowspec(NB, HID), _rowspec(NB, HID)],
        out_shape=[jax.ShapeDtypeStruct((N, HID), f32),
                   jax.ShapeDtypeStruct((N, HID), f32),
                   jax.ShapeDtypeStruct((N, HID), jnp.int32)],
    )(nf_pad, Wv, bv, Wq, bq, Wk, bk, Wvv, bvv)


def _edge_embed_body(ef_r, d_r, We_r, be_r, e_r, env_r):
    e_r[...] = (_dot(ef_r[...], We_r[...]) + be_r[...]).astype(jnp.bfloat16)
    d = d_r[...]
    x01 = jnp.clip(d / CUT, 0.0, 1.0)
    x2 = x01 * x01
    x3 = x2 * x01
    x4 = x2 * x2
    x5 = x4 * x01
    env_r[...] = jnp.where(d < CUT, 1.0 - 6.0 * x5 + 15.0 * x4 - 10.0 * x3, 0.0)


def _edge_embed(ef_pad, dist, We, be):
    return pl.pallas_call(
        _edge_embed_body, grid=(E // EB,),
        in_specs=[_rowspec(EB, 40), _rowspec(EB, 1),
                  _fullspec((40, HID)), _fullspec((1, HID))],
        out_specs=[_rowspec(EB, HID), _rowspec(EB, 1)],
        out_shape=[jax.ShapeDtypeStruct((E, HID), jnp.bfloat16),
                   jax.ShapeDtypeStruct((E, 1), f32)],
    )(ef_pad, dist, We, be)


def _edge_attn_body(e_r, ksvs_r, qd_r, env_r, Wee_r, bee_r,
                    score_r, contrib_r, arep_r):
    e = e_r[...].astype(f32)
    w = lax.bitcast_convert_type(ksvs_r[...], jnp.uint32)
    ks = lax.bitcast_convert_type(
        lax.convert_element_type(w, jnp.uint16), jnp.bfloat16).astype(f32)
    vs = lax.bitcast_convert_type(
        lax.convert_element_type(w >> 16, jnp.uint16),
        jnp.bfloat16).astype(f32)
    qd = qd_r[...]
    ee = _dot(e, Wee_r[...]) + bee_r[...]
    score = ks * qd * (0.25 * ee)
    s = jnp.clip(_dot(score, _sel_hd()), -5.0, 5.0)
    a = jnp.exp(s) * env_r[...]
    arep = _dot(a, _sel_dh())
    contrib_r[...] = arep * vs
    arep_r[...] = arep
    score_r[...] = score.astype(jnp.bfloat16)


def _edge_attn(e, ksvs, qd, env, Wee, bee):
    return pl.pallas_call(
        _edge_attn_body, grid=(E // EB,),
        in_specs=[_rowspec(EB, HID), _rowspec(EB, HID),
                  _rowspec(EB, HID), _rowspec(EB, 1),
                  _fullspec((HID, HID)), _fullspec((1, HID))],
        out_specs=[_rowspec(EB, HID), _rowspec(EB, HID), _rowspec(EB, HID)],
        out_shape=[jax.ShapeDtypeStruct((E, HID), jnp.bfloat16),
                   jax.ShapeDtypeStruct((E, HID), f32),
                   jax.ShapeDtypeStruct((E, HID), f32)],
    )(e, ksvs, qd, env, Wee, bee)


def _edge_up_body(e_r, score_r, Woe_r, boe_r, g1_r, b1_r,
                  Wf1_r, bf1_r, Wf2_r, bf2_r, g2_r, b2_r, e2_r):
    e = e_r[...].astype(f32)
    score = score_r[...].astype(f32)
    eh = e + _dot(score, Woe_r[...]) + boe_r[...]
    eh = _lnk(eh, g1_r[...], b1_r[...])
    h = jnp.maximum(_dot(eh, Wf1_r[...]) + bf1_r[...], 0.0)
    e2_r[...] = _lnk(eh + _dot(h, Wf2_r[...]) + bf2_r[...], g2_r[...],
                     b2_r[...]).astype(jnp.bfloat16)


def _edge_up(e, score, Woe, boe, g1, b1, Wf1, bf1, Wf2, bf2, g2, b2):
    return pl.pallas_call(
        _edge_up_body, grid=(E // EB,),
        in_specs=[_rowspec(EB, HID), _rowspec(EB, HID),
                  _fullspec((HID, HID)), _fullspec((1, HID)),
                  _fullspec((1, HID)), _fullspec((1, HID)),
                  _fullspec((HID, FFN)), _fullspec((1, FFN)),
                  _fullspec((FFN, HID)), _fullspec((1, HID)),
                  _fullspec((1, HID)), _fullspec((1, HID))],
        out_specs=_rowspec(EB, HID),
        out_shape=jax.ShapeDtypeStruct((E, HID), jnp.bfloat16),
    )(e, score, Woe, boe, g1, b1, Wf1, bf1, Wf2, bf2, g2, b2)


def _node_common(v_r, aggp_r, denp_r, Wo_r, bo_r, g1_r, b1_r,
                 Wf1_r, bf1_r, Wf2_r, bf2_r, g2_r, b2_r):
    agg = aggp_r[0] + aggp_r[1]
    denr = denp_r[0] + denp_r[1] + 1e-6
    v_att = agg / denr
    vh = v_r[...] + _dot(v_att, Wo_r[...]) + bo_r[...]
    vh = _lnk(vh, g1_r[...], b1_r[...])
    h = jnp.maximum(_dot(vh, Wf1_r[...]) + bf1_r[...], 0.0)
    return _lnk(vh + _dot(h, Wf2_r[...]) + bf2_r[...], g2_r[...], b2_r[...])


def _node_layer_body(v_r, aggp_r, denp_r, Wo_r, bo_r, g1_r, b1_r,
                     Wf1_r, bf1_r, Wf2_r, bf2_r, g2_r, b2_r,
                     Wq_r, bq_r, Wk_r, bk_r, Wvv_r, bvv_r,
                     v2_r, q_r, kv_r):
    v2 = _node_common(v_r, aggp_r, denp_r, Wo_r, bo_r, g1_r, b1_r,
                      Wf1_r, bf1_r, Wf2_r, bf2_r, g2_r, b2_r)
    v2_r[...] = v2
    q_r[...] = _dot(v2, Wq_r[...]) + bq_r[...]
    k = _dot(v2, Wk_r[...]) + bk_r[...]
    val = _dot(v2, Wvv_r[...]) + bvv_r[...]
    kh = lax.convert_element_type(
        lax.bitcast_convert_type(k.astype(jnp.bfloat16), jnp.uint16),
        jnp.uint32)
    vh = lax.convert_element_type(
        lax.bitcast_convert_type(val.astype(jnp.bfloat16), jnp.uint16),
        jnp.uint32)
    kv_r[...] = lax.bitcast_convert_type(kh | (vh << 16), jnp.int32)


def _node_layer(v, aggp, denp, Wo, bo, g1, b1, Wf1, bf1, Wf2, bf2, g2, b2,
                Wq, bq, Wk, bk, Wvv, bvv):
    wspecs = [_fullspec((HID, HID)), _fullspec((1, HID)),
              _fullspec((1, HID)), _fullspec((1, HID)),
              _fullspec((HID, FFN)), _fullspec((1, FFN)),
              _fullspec((FFN, HID)), _fullspec((1, HID)),
              _fullspec((1, HID)), _fullspec((1, HID))]
    qkvspecs = [_fullspec((HID, HID)), _fullspec((1, HID)),
                _fullspec((HID, HID)), _fullspec((1, HID)),
                _fullspec((HID, HID)), _fullspec((1, HID))]
    return pl.pallas_call(
        _node_layer_body, grid=(N // NB,),
        in_specs=[_rowspec(NB, HID),
                  pl.BlockSpec((2, NB, HID), lambda i: (0, i, 0)),
                  pl.BlockSpec((2, NB, HID), lambda i: (0, i, 0))]
                 + wspecs + qkvspecs,
        out_specs=[_rowspec(NB, HID), _rowspec(NB, HID), _rowspec(NB, HID)],
        out_shape=[jax.ShapeDtypeStruct((N, HID), f32),
                   jax.ShapeDtypeStruct((N, HID), f32),
                   jax.ShapeDtypeStruct((N, HID), jnp.int32)],
    )(v, aggp, denp, Wo, bo, g1, b1, Wf1, bf1, Wf2, bf2, g2, b2,
      Wq, bq, Wk, bk, Wvv, bvv)


def _node_final_body(v_r, aggp_r, denp_r, nf0_r, rad_r,
                     Wo_r, bo_r, g1_r, b1_r, Wf1_r, bf1_r, Wf2_r, bf2_r,
                     g2_r, b2_r, vfin_r, z_r):
    v2 = _node_common(v_r, aggp_r, denp_r, Wo_r, bo_r, g1_r, b1_r,
                      Wf1_r, bf1_r, Wf2_r, bf2_r, g2_r, b2_r)
    vsc = v2 * nf0_r[...]
    vfin_r[:, :HID] = vsc
    lane = lax.broadcasted_iota(jnp.int32, (1, HID), 1)
    vfin_r[:, HID:] = jnp.where(lane == 0, rad_r[...], 0.0)

    @pl.when(pl.program_id(0) == 0)
    def _():
        z_r[...] = jnp.zeros_like(z_r)

    z_r[...] += jnp.sum(vsc, axis=0, keepdims=True)


def _node_final(v, aggp, denp, nf0, rad, Wo, bo, g1, b1, Wf1, bf1, Wf2, bf2,
                g2, b2):
    wspecs = [_fullspec((HID, HID)), _fullspec((1, HID)),
              _fullspec((1, HID)), _fullspec((1, HID)),
              _fullspec((HID, FFN)), _fullspec((1, FFN)),
              _fullspec((FFN, HID)), _fullspec((1, HID)),
              _fullspec((1, HID)), _fullspec((1, HID))]
    return pl.pallas_call(
        _node_final_body, grid=(N // NB,),
        in_specs=[_rowspec(NB, HID),
                  pl.BlockSpec((2, NB, HID), lambda i: (0, i, 0)),
                  pl.BlockSpec((2, NB, HID), lambda i: (0, i, 0)),
                  _rowspec(NB, 1), _rowspec(NB, 1)] + wspecs,
        out_specs=[_rowspec(NB, 2 * HID), _fullspec((1, HID))],
        out_shape=[jax.ShapeDtypeStruct((N, 2 * HID), f32),
                   jax.ShapeDtypeStruct((1, HID), f32)],
    )(v, aggp, denp, nf0, rad, Wo, bo, g1, b1, Wf1, bf1, Wf2, bf2, g2, b2)


def _sigmoid(x):
    return 1.0 / (1.0 + jnp.exp(-x))


def _readout_body(xs_r, xd_r, d_r, it_r, ef0_r,
                  W21_r, b21_r, W22_r, b22_r, W31_r, b31_r, W32_r, b32_r,
                  y_r):
    x = xs_r[:, :HID] + xd_r[:, :HID]
    rsum = xs_r[:, HID:HID + 1] + xd_r[:, HID:HID + 1]
    dist = d_r[...]
    d = dist - rsum
    t = d * 1.25
    V0 = -0.045 * jnp.exp(-(t * t))
    V1 = 0.8 * jnp.where(d < 0, d * d, 0.0)
    V2 = -0.035 * (jnp.where((d > 0) & (d < 2.5), -0.4 * (d - 2.5), 0.0)
                   + jnp.where(d <= 0, 1.0, 0.0))
    V3 = -0.6 * (jnp.where((d > -0.6) & (d < 0), (-5.0 / 3.0) * d, 0.0)
                 + jnp.where(d <= -0.6, 1.0, 0.0))
    mask = jnp.where(dist < CUT, 1.0, 0.0) * ef0_r[...]
    w2 = _sigmoid(_dot(jnp.maximum(_dot(x, W21_r[...]) + b21_r[...], 0.0),
                       W22_r[...]) + b22_r[...]) + 0.5
    w3 = _sigmoid(_dot(jnp.maximum(_dot(x, W31_r[...]) + b31_r[...], 0.0),
                       W32_r[...]) + b32_r[...]) + 0.5
    it1 = it_r[:, 1:2]
    it2 = it_r[:, 2:3]
    t2 = (w2[:, 0:1] * V0 + w2[:, 1:2] * V1 + it1 * w2[:, 2:3] * V2
          + it2 * w2[:, 3:4] * V3)
    t3 = (w3[:, 0:1] * V0 + w3[:, 1:2] * V1 + it1 * w3[:, 2:3] * V2
          + it2 * w3[:, 3:4] * V3)
    p2 = jnp.sum(mask * t2) * 0.5
    p3 = jnp.sum(mask * t3) * 0.5

    @pl.when(pl.program_id(0) == 0)
    def _():
        y_r[...] = jnp.zeros_like(y_r)

    lane = lax.broadcasted_iota(jnp.int32, (1, HID), 1)
    y_r[...] += jnp.where(lane == 0, p2, 0.0) + jnp.where(lane == 1, p3, 0.0)


def _readout(xs, xd, dist, itp, ef0, W21, b21, W22, b22, W31, b31, W32, b32):
    return pl.pallas_call(
        _readout_body, grid=(E // EB,),
        in_specs=[_rowspec(EB, 2 * HID), _rowspec(EB, 2 * HID), _rowspec(EB, 1),
                  _rowspec(EB, 8), _rowspec(EB, 1),
                  _fullspec((HID, HID)), _fullspec((1, HID)),
                  _fullspec((HID, 8)), _fullspec((1, 8)),
                  _fullspec((HID, HID)), _fullspec((1, HID)),
                  _fullspec((HID, 8)), _fullspec((1, 8))],
        out_specs=_fullspec((1, HID)),
        out_shape=jax.ShapeDtypeStruct((1, HID), f32),
    )(xs, xd, dist, itp, ef0, W21, b21, W22, b22, W31, b31, W32, b32)


def _head_body(z_r, y_r, nrot_r, W11_r, b11_r, W12_r, b12_r,
               W41_r, b41_r, W42_r, b42_r, out_r):
    z = z_r[...]
    y1 = (_dot(jnp.maximum(_dot(z, W11_r[...]) + b11_r[...], 0.0),
               W12_r[...]) + b12_r[...])[0, 0]
    w4 = _sigmoid((_dot(jnp.maximum(_dot(z, W41_r[...]) + b41_r[...], 0.0),
                        W42_r[...]) + b42_r[...])[0, 0]) + 0.5
    y2 = y_r[0, 0]
    y3 = y_r[0, 1] / (1.0 + w4 * 0.05846 * nrot_r[0, 0])
    lane = lax.broadcasted_iota(jnp.int32, (1, HID), 1)
    out_r[...] = (jnp.where(lane == 0, y1, 0.0)
                  + jnp.where(lane == 1, y2, 0.0)
                  + jnp.where(lane == 2, y3, 0.0))


def _head(z, y, nrot, W11, b11, W12, b12, W41, b41, W42, b42):
    return pl.pallas_call(
        _head_body,
        out_shape=jax.ShapeDtypeStruct((1, HID), f32),
    )(z, y, nrot, W11, b11, W12, b12, W41, b41, W42, b42)


def kernel(node_feature, edge_feature, vdw_radii, distance, interaction_type,
           edge_index, n_rot, params):
    p = params
    src = edge_index[0].astype(jnp.int32)
    dst = edge_index[1].astype(jnp.int32)
    nf_pad = jnp.pad(node_feature, ((0, 0), (0, 1)))
    ef_pad = jnp.pad(edge_feature, ((0, 0), (0, 1)))
    Wv = jnp.pad(p['Wv_emb'], ((0, 1), (0, 0)))
    We = jnp.pad(p['We_emb'], ((0, 1), (0, 0)))
    dist = distance[:, None]
    itp = jnp.pad(interaction_type, ((0, 0), (0, 5)))
    ef0 = edge_feature[:, 0:1]
    nf0 = node_feature[:, 0:1]
    rad = vdw_radii[:, None]
    nrot = jnp.asarray(n_rot, f32).reshape(1, 1)

    def row(x):
        return x.reshape(1, -1)

    v, q_t, kv_t = _node_embed(nf_pad, Wv, row(p['bv_emb']),
                               p['Wq'][0], row(p['bq'][0]),
                               p['Wk'][0], row(p['bk'][0]),
                               p['Wvv'][0], row(p['bvv'][0]))
    e, env = _edge_embed(ef_pad, dist, We, row(p['be_emb']))

    vfin = None
    z = None
    for l in range(NLAYERS):
        ksvs, qd = _gather_layer(kv_t, q_t, src, dst)
        score, contrib, arep = _edge_attn(e, ksvs, qd, env,
                                          p['Wee'][l], row(p['bee'][l]))
        aggf, denf = _scatter(contrib, arep, dst)
        e = _edge_up(e, score, p['Woe'][l], row(p['boe'][l]),
                     row(p['g1e'][l]), row(p['b1e'][l]),
                     p['Wf1e'][l], row(p['bf1e'][l]),
                     p['Wf2e'][l], row(p['bf2e'][l]),
                     row(p['g2e'][l]), row(p['b2e'][l]))
        aggp = aggf.reshape(2, NPAD, HID)
        denp = denf.reshape(2, NPAD, HID)
        nw = (p['Wo'][l], row(p['bo'][l]), row(p['g1v'][l]), row(p['b1v'][l]),
              p['Wf1v'][l], row(p['bf1v'][l]), p['Wf2v'][l], row(p['bf2v'][l]),
              row(p['g2v'][l]), row(p['b2v'][l]))
        if l < NLAYERS - 1:
            v, q_t, kv_t = _node_layer(v, aggp, denp, *nw,
                                       p['Wq'][l + 1], row(p['bq'][l + 1]),
                                       p['Wk'][l + 1], row(p['bk'][l + 1]),
                                       p['Wvv'][l + 1], row(p['bvv'][l + 1]))
        else:
            vfin, z = _node_final(v, aggp, denp, nf0, rad, *nw)

    xs, xd = _gather_fin(vfin, vfin, src, dst)
    y23 = _readout(xs, xd, dist, itp, ef0,
                   p['r2_W1'], row(p['r2_b1']),
                   jnp.pad(p['r2_W2'], ((0, 0), (0, 4))),
                   row(jnp.pad(p['r2_b2'], (0, 4))),
                   p['r3_W1'], row(p['r3_b1']),
                   jnp.pad(p['r3_W2'], ((0, 0), (0, 4))),
                   row(jnp.pad(p['r3_b2'], (0, 4))))
    out = _head(z, y23, nrot,
                p['r1_W1'], row(p['r1_b1']),
                jnp.pad(p['r1_W2'], ((0, 0), (0, 7))),
                row(jnp.pad(p['r1_b2'], (0, 7))),
                p['r4_W1'], row(p['r4_b1']),
                jnp.pad(p['r4_W2'], ((0, 0), (0, 7))),
                row(jnp.pad(p['r4_b2'], (0, 7))))
    return out[0, :3]


# speedup vs baseline: 35.7376x; 1.0147x over previous
import functools

import jax
import jax.numpy as jnp
from jax import lax
from jax.experimental import pallas as pl
from jax.experimental.pallas import tpu as pltpu
from jax.experimental.pallas import tpu_sc as plsc

N = 10000
E = 160000
HID = 128
HEADS = 8
DH = HID // HEADS
NLAYERS = 10
FFN = 2 * HID
CUT = 6.5
f32 = jnp.float32

NC = 2
NS = 16
NW = NC * NS
PER_W = E // NW
CH = 104
NFULL = PER_W // CH
NPAIR = NFULL // 2
TAIL = PER_W - NFULL * CH
DCH = 128
NPAD = 10240
TROWS = NPAD // NS
ZR = 128

EB = 2000
NB = 1000

_PREC = jax.lax.Precision.DEFAULT


def _dot(a, b):
    return jnp.dot(a, b, preferred_element_type=f32, precision=_PREC)


def _lnk(x, g, b):
    m = jnp.mean(x, axis=-1, keepdims=True)
    v = jnp.mean((x - m) * (x - m), axis=-1, keepdims=True)
    return (x - m) * lax.rsqrt(v + 1e-5) * g + b


def _sel_hd():
    r = lax.broadcasted_iota(jnp.int32, (HID, HEADS), 0) // DH
    c = lax.broadcasted_iota(jnp.int32, (HID, HEADS), 1)
    return (r == c).astype(f32)


def _sel_dh():
    r = lax.broadcasted_iota(jnp.int32, (HEADS, HID), 0)
    c = lax.broadcasted_iota(jnp.int32, (HEADS, HID), 1) // DH
    return (r == c).astype(f32)


def _pack64(x):
    lo = lax.convert_element_type(
        lax.bitcast_convert_type(x[:, :64].astype(jnp.bfloat16), jnp.uint16),
        jnp.uint32)
    hi = lax.convert_element_type(
        lax.bitcast_convert_type(x[:, 64:].astype(jnp.bfloat16), jnp.uint16),
        jnp.uint32)
    return lax.bitcast_convert_type(lo | (hi << 16), jnp.int32)


def _unpack64(w):
    u = lax.bitcast_convert_type(w, jnp.uint32)
    lo = lax.bitcast_convert_type(
        lax.convert_element_type(u, jnp.uint16), jnp.bfloat16)
    hi = lax.bitcast_convert_type(
        lax.convert_element_type(u >> 16, jnp.uint16), jnp.bfloat16)
    return jnp.concatenate([lo, hi], axis=1).astype(f32)


def _rowspec(b, w):
    return pl.BlockSpec((b, w), lambda i: (i, 0))


def _fullspec(shape):
    return pl.BlockSpec(shape, lambda i: tuple(0 for _ in shape))


@functools.lru_cache(maxsize=None)
def _make_gather(rsa, dta, rsb, dtb):
    mesh = plsc.VectorSubcoreMesh(core_axis_name="c", subcore_axis_name="s")

    @functools.partial(
        pl.kernel, mesh=mesh,
        out_type=(jax.ShapeDtypeStruct((E,) + rsa, dta),
                  jax.ShapeDtypeStruct((E,) + rsb, dtb)),
        scratch_types=(
            pltpu.VMEM((2, CH), jnp.int32), pltpu.VMEM((2, CH), jnp.int32),
            pltpu.VMEM((TAIL,), jnp.int32), pltpu.VMEM((TAIL,), jnp.int32),
            pltpu.VMEM((2, CH) + rsa, dta), pltpu.VMEM((2, CH) + rsb, dtb),
            pltpu.VMEM((TAIL,) + rsa, dta), pltpu.VMEM((TAIL,) + rsb, dtb),
            pltpu.SemaphoreType.DMA, pltpu.SemaphoreType.DMA,
            pltpu.SemaphoreType.DMA, pltpu.SemaphoreType.DMA,
            pltpu.SemaphoreType.DMA, pltpu.SemaphoreType.DMA,
        ),
    )
    def gk(tabA, tabB, idxA, idxB, outA, outB,
           ia, ib, ta, tb, ra, rb, tra, trb,
           sga, sgb, swa0, swb0, swa1, swb1):
        wid = lax.axis_index("s") * NC + lax.axis_index("c")
        base = wid * PER_W
        sw = ((swa0, swb0), (swa1, swb1))

        def pair(g, carry):
            @pl.when(g > 0)
            def _():
                for b in range(2):
                    pltpu.make_async_copy(ra.at[b], outA.at[pl.ds(0, CH)],
                                          sw[b][0]).wait()
                    pltpu.make_async_copy(rb.at[b], outB.at[pl.ds(0, CH)],
                                          sw[b][1]).wait()

            copies = []
            for b in range(2):
                off = base + (g * 2 + b) * CH
                pltpu.sync_copy(idxA.at[pl.ds(off, CH)], ia.at[b])
                pltpu.sync_copy(idxB.at[pl.ds(off, CH)], ib.at[b])
                copies.append(
                    (pltpu.async_copy(tabA.at[ia.at[b]], ra.at[b], sga),
                     pltpu.async_copy(tabB.at[ib.at[b]], rb.at[b], sgb)))
            for b in range(2):
                off = base + (g * 2 + b) * CH
                ca, cb = copies[b]
                ca.wait()
                cb.wait()
                pltpu.async_copy(ra.at[b], outA.at[pl.ds(off, CH)], sw[b][0])
                pltpu.async_copy(rb.at[b], outB.at[pl.ds(off, CH)], sw[b][1])
            return carry

        lax.fori_loop(0, NPAIR, pair, 0)
        for b in range(2):
            pltpu.make_async_copy(ra.at[b], outA.at[pl.ds(0, CH)],
                                  sw[b][0]).wait()
            pltpu.make_async_copy(rb.at[b], outB.at[pl.ds(0, CH)],
                                  sw[b][1]).wait()
        off = base + NFULL * CH
        pltpu.sync_copy(idxA.at[pl.ds(off, TAIL)], ta)
        pltpu.sync_copy(idxB.at[pl.ds(off, TAIL)], tb)
        ca = pltpu.async_copy(tabA.at[ta], tra, sga)
        cb = pltpu.async_copy(tabB.at[tb], trb, sgb)
        ca.wait()
        cb.wait()
        pltpu.sync_copy(tra, outA.at[pl.ds(off, TAIL)])
        pltpu.sync_copy(trb, outB.at[pl.ds(off, TAIL)])

    return gk


def _gather_layer(kv_t, q_t, src, dst):
    return _make_gather((HID,), jnp.int32, (HID,), f32)(kv_t, q_t, src, dst)


def _gather_fin(vfin_a, vfin_b, src, dst):
    return _make_gather((HID,), jnp.int32, (HID,), jnp.int32)(vfin_a, vfin_b,
                                                              src, dst)


@functools.lru_cache(maxsize=None)
def _make_scatter():
    mesh = plsc.VectorSubcoreMesh(core_axis_name="c", subcore_axis_name="s")

    @functools.partial(
        pl.kernel, mesh=mesh,
        out_type=(jax.ShapeDtypeStruct((2 * NPAD, HID), f32),
                  jax.ShapeDtypeStruct((2 * NPAD, HID), f32)),
        scratch_types=(
            pltpu.VMEM((2, CH), jnp.int32),
            pltpu.VMEM((2, CH, HID), f32),
            pltpu.VMEM((TAIL,), jnp.int32),
            pltpu.VMEM((TAIL, HID), f32),
            pltpu.VMEM((ZR, HID), f32),
            pltpu.VMEM_SHARED((NPAD, HID), f32),
            pltpu.SemaphoreType.DMA, pltpu.SemaphoreType.DMA,
        ),
    )
    def sck(contrib_h, arep_h, dst_h, agg_out, den_out,
            idx_v, rows_v, idx_t, rows_t, zb, acc_sh, sl0, sl1):
        c = lax.axis_index("c")
        s = lax.axis_index("s")
        zeros16 = jnp.zeros((16,), f32)

        def zero_zb():
            def zrow(r, carry):
                def zcol(j, cc):
                    zb[r, pl.ds(j * 16, 16)] = zeros16
                    return cc
                return lax.fori_loop(0, HID // 16, zcol, carry)

            lax.fori_loop(0, ZR, zrow, 0)

        zero_zb()

        r0 = s * TROWS
        wid = s * NC + c
        base = wid * PER_W
        sl = (sl0, sl1)

        def zero_acc():
            for b in range(TROWS // ZR):
                pltpu.sync_copy(zb, acc_sh.at[pl.ds(r0 + b * ZR, ZR)])

        def scatter_pass(src_h, out_h):
            def pair(g, carry):
                copies = []
                for b in range(2):
                    off = base + (g * 2 + b) * CH
                    pltpu.sync_copy(dst_h.at[pl.ds(off, CH)], idx_v.at[b])
                    copies.append(pltpu.async_copy(
                        src_h.at[pl.ds(off, CH)], rows_v.at[b], sl[b]))
                for b in range(2):
                    copies[b].wait()
                    pltpu.sync_copy(rows_v.at[b], acc_sh.at[idx_v.at[b]],
                                    add=True)
                return carry

            lax.fori_loop(0, NPAIR, pair, 0)
            off = base + NFULL * CH
            pltpu.sync_copy(dst_h.at[pl.ds(off, TAIL)], idx_t)
            pltpu.sync_copy(src_h.at[pl.ds(off, TAIL)], rows_t)
            pltpu.sync_copy(rows_t, acc_sh.at[idx_t], add=True)
            plsc.subcore_barrier()
            for b in range(TROWS // DCH):
                pltpu.sync_copy(acc_sh.at[pl.ds(r0 + b * DCH, DCH)], zb)
                pltpu.sync_copy(zb,
                                out_h.at[pl.ds(c * NPAD + r0 + b * DCH, DCH)])
            plsc.subcore_barrier()

        zero_acc()
        plsc.subcore_barrier()
        scatter_pass(contrib_h, agg_out)
        zero_zb()
        zero_acc()
        plsc.subcore_barrier()
        scatter_pass(arep_h, den_out)

    return sck


def _scatter(contrib, arep, dst):
    return _make_scatter()(contrib, arep, dst)


def _node_embed_body(nf_r, Wv_r, bv_r, Wq_r, bq_r, Wk_r, bk_r, Wvv_r, bvv_r,
                     v_r, q_r, kv_r):
    v = _dot(nf_r[...], Wv_r[...]) + bv_r[...]
    v_r[...] = v
    q_r[...] = _dot(v, Wq_r[...]) + bq_r[...]
    k = _dot(v, Wk_r[...]) + bk_r[...]
    val = _dot(v, Wvv_r[...]) + bvv_r[...]
    kh = lax.convert_element_type(
        lax.bitcast_convert_type(k.astype(jnp.bfloat16), jnp.uint16),
        jnp.uint32)
    vh = lax.convert_element_type(
        lax.bitcast_convert_type(val.astype(jnp.bfloat16), jnp.uint16),
        jnp.uint32)
    kv_r[...] = lax.bitcast_convert_type(kh | (vh << 16), jnp.int32)


def _node_embed(nf_pad, Wv, bv, Wq, bq, Wk, bk, Wvv, bvv):
    return pl.pallas_call(
        _node_embed_body, grid=(N // NB,),
        in_specs=[_rowspec(NB, 40), _fullspec((40, HID)), _fullspec((1, HID)),
                  _fullspec((HID, HID)), _fullspec((1, HID)),
                  _fullspec((HID, HID)), _fullspec((1, HID)),
                  _fullspec((HID, HID)), _fullspec((1, HID))],
        out_specs=[_rowspec(NB, HID), _rowspec(NB, HID), _rowspec(NB, HID)],
        out_shape=[jax.ShapeDtypeStruct((N, HID), f32),
                   jax.ShapeDtypeStruct((N, HID), f32),
                   jax.ShapeDtypeStruct((N, HID), jnp.int32)],
    )(nf_pad, Wv, bv, Wq, bq, Wk, bk, Wvv, bvv)


def _edge_embed_body(ef_r, d_r, We_r, be_r, e_r, env_r):
    e_r[...] = (_dot(ef_r[...], We_r[...]) + be_r[...]).astype(jnp.bfloat16)
    d = d_r[...]
    x01 = jnp.clip(d / CUT, 0.0, 1.0)
    x2 = x01 * x01
    x3 = x2 * x01
    x4 = x2 * x2
    x5 = x4 * x01
    env_r[...] = jnp.where(d < CUT, 1.0 - 6.0 * x5 + 15.0 * x4 - 10.0 * x3, 0.0)


def _edge_embed(ef_pad, dist, We, be):
    return pl.pallas_call(
        _edge_embed_body, grid=(E // EB,),
        in_specs=[_rowspec(EB, 40), _rowspec(EB, 1),
                  _fullspec((40, HID)), _fullspec((1, HID))],
        out_specs=[_rowspec(EB, HID), _rowspec(EB, 1)],
        out_shape=[jax.ShapeDtypeStruct((E, HID), jnp.bfloat16),
                   jax.ShapeDtypeStruct((E, 1), f32)],
    )(ef_pad, dist, We, be)


def _edge_attn_body(e_r, ksvs_r, qd_r, env_r, Wee_r, bee_r,
                    score_r, contrib_r, arep_r):
    e = e_r[...].astype(f32)
    w = lax.bitcast_convert_type(ksvs_r[...], jnp.uint32)
    ks = lax.bitcast_convert_type(
        lax.convert_element_type(w, jnp.uint16), jnp.bfloat16).astype(f32)
    vs = lax.bitcast_convert_type(
        lax.convert_element_type(w >> 16, jnp.uint16),
        jnp.bfloat16).astype(f32)
    qd = qd_r[...]
    ee = _dot(e, Wee_r[...]) + bee_r[...]
    score = ks * qd * (0.25 * ee)
    s = jnp.clip(_dot(score, _sel_hd()), -5.0, 5.0)
    a = jnp.exp(s) * env_r[...]
    arep = _dot(a, _sel_dh())
    contrib_r[...] = arep * vs
    arep_r[...] = arep
    score_r[...] = score.astype(jnp.bfloat16)


def _edge_attn(e, ksvs, qd, env, Wee, bee):
    return pl.pallas_call(
        _edge_attn_body, grid=(E // EB,),
        in_specs=[_rowspec(EB, HID), _rowspec(EB, HID),
                  _rowspec(EB, HID), _rowspec(EB, 1),
                  _fullspec((HID, HID)), _fullspec((1, HID))],
        out_specs=[_rowspec(EB, HID), _rowspec(EB, HID), _rowspec(EB, HID)],
        out_shape=[jax.ShapeDtypeStruct((E, HID), jnp.bfloat16),
                   jax.ShapeDtypeStruct((E, HID), f32),
                   jax.ShapeDtypeStruct((E, HID), f32)],
    )(e, ksvs, qd, env, Wee, bee)


def _edge_up_body(e_r, score_r, Woe_r, boe_r, g1_r, b1_r,
                  Wf1_r, bf1_r, Wf2_r, bf2_r, g2_r, b2_r, e2_r):
    e = e_r[...].astype(f32)
    score = score_r[...].astype(f32)
    eh = e + _dot(score, Woe_r[...]) + boe_r[...]
    eh = _lnk(eh, g1_r[...], b1_r[...])
    h = jnp.maximum(_dot(eh, Wf1_r[...]) + bf1_r[...], 0.0)
    e2_r[...] = _lnk(eh + _dot(h, Wf2_r[...]) + bf2_r[...], g2_r[...],
                     b2_r[...]).astype(jnp.bfloat16)


def _edge_up(e, score, Woe, boe, g1, b1, Wf1, bf1, Wf2, bf2, g2, b2):
    return pl.pallas_call(
        _edge_up_body, grid=(E // EB,),
        in_specs=[_rowspec(EB, HID), _rowspec(EB, HID),
                  _fullspec((HID, HID)), _fullspec((1, HID)),
                  _fullspec((1, HID)), _fullspec((1, HID)),
                  _fullspec((HID, FFN)), _fullspec((1, FFN)),
                  _fullspec((FFN, HID)), _fullspec((1, HID)),
                  _fullspec((1, HID)), _fullspec((1, HID))],
        out_specs=_rowspec(EB, HID),
        out_shape=jax.ShapeDtypeStruct((E, HID), jnp.bfloat16),
    )(e, score, Woe, boe, g1, b1, Wf1, bf1, Wf2, bf2, g2, b2)


def _node_common(v_r, aggp_r, denp_r, Wo_r, bo_r, g1_r, b1_r,
                 Wf1_r, bf1_r, Wf2_r, bf2_r, g2_r, b2_r):
    agg = aggp_r[0] + aggp_r[1]
    denr = denp_r[0] + denp_r[1] + 1e-6
    v_att = agg / denr
    vh = v_r[...] + _dot(v_att, Wo_r[...]) + bo_r[...]
    vh = _lnk(vh, g1_r[...], b1_r[...])
    h = jnp.maximum(_dot(vh, Wf1_r[...]) + bf1_r[...], 0.0)
    return _lnk(vh + _dot(h, Wf2_r[...]) + bf2_r[...], g2_r[...], b2_r[...])


def _node_layer_body(v_r, aggp_r, denp_r, Wo_r, bo_r, g1_r, b1_r,
                     Wf1_r, bf1_r, Wf2_r, bf2_r, g2_r, b2_r,
                     Wq_r, bq_r, Wk_r, bk_r, Wvv_r, bvv_r,
                     v2_r, q_r, kv_r):
    v2 = _node_common(v_r, aggp_r, denp_r, Wo_r, bo_r, g1_r, b1_r,
                      Wf1_r, bf1_r, Wf2_r, bf2_r, g2_r, b2_r)
    v2_r[...] = v2
    q_r[...] = _dot(v2, Wq_r[...]) + bq_r[...]
    k = _dot(v2, Wk_r[...]) + bk_r[...]
    val = _dot(v2, Wvv_r[...]) + bvv_r[...]
    kh = lax.convert_element_type(
        lax.bitcast_convert_type(k.astype(jnp.bfloat16), jnp.uint16),
        jnp.uint32)
    vh = lax.convert_element_type(
        lax.bitcast_convert_type(val.astype(jnp.bfloat16), jnp.uint16),
        jnp.uint32)
    kv_r[...] = lax.bitcast_convert_type(kh | (vh << 16), jnp.int32)


def _node_layer(v, aggp, denp, Wo, bo, g1, b1, Wf1, bf1, Wf2, bf2, g2, b2,
                Wq, bq, Wk, bk, Wvv, bvv):
    wspecs = [_fullspec((HID, HID)), _fullspec((1, HID)),
              _fullspec((1, HID)), _fullspec((1, HID)),
              _fullspec((HID, FFN)), _fullspec((1, FFN)),
              _fullspec((FFN, HID)), _fullspec((1, HID)),
              _fullspec((1, HID)), _fullspec((1, HID))]
    qkvspecs = [_fullspec((HID, HID)), _fullspec((1, HID)),
                _fullspec((HID, HID)), _fullspec((1, HID)),
                _fullspec((HID, HID)), _fullspec((1, HID))]
    return pl.pallas_call(
        _node_layer_body, grid=(N // NB,),
        in_specs=[_rowspec(NB, HID),
                  pl.BlockSpec((2, NB, HID), lambda i: (0, i, 0)),
                  pl.BlockSpec((2, NB, HID), lambda i: (0, i, 0))]
                 + wspecs + qkvspecs,
        out_specs=[_rowspec(NB, HID), _rowspec(NB, HID), _rowspec(NB, HID)],
        out_shape=[jax.ShapeDtypeStruct((N, HID), f32),
                   jax.ShapeDtypeStruct((N, HID), f32),
                   jax.ShapeDtypeStruct((N, HID), jnp.int32)],
    )(v, aggp, denp, Wo, bo, g1, b1, Wf1, bf1, Wf2, bf2, g2, b2,
      Wq, bq, Wk, bk, Wvv, bvv)


def _node_final_body(v_r, aggp_r, denp_r, nf0_r, rad_r,
                     Wo_r, bo_r, g1_r, b1_r, Wf1_r, bf1_r, Wf2_r, bf2_r,
                     g2_r, b2_r, vfin_r, z_r):
    v2 = _node_common(v_r, aggp_r, denp_r, Wo_r, bo_r, g1_r, b1_r,
                      Wf1_r, bf1_r, Wf2_r, bf2_r, g2_r, b2_r)
    vsc = v2 * nf0_r[...]
    lane = lax.broadcasted_iota(jnp.int32, (1, 64), 1)
    radw = jnp.where(lane == 0,
                     lax.bitcast_convert_type(rad_r[...], jnp.int32), 0)
    vfin_r[...] = jnp.concatenate([_pack64(vsc), radw], axis=1)

    @pl.when(pl.program_id(0) == 0)
    def _():
        z_r[...] = jnp.zeros_like(z_r)

    z_r[...] += jnp.sum(vsc, axis=0, keepdims=True)


def _node_final(v, aggp, denp, nf0, rad, Wo, bo, g1, b1, Wf1, bf1, Wf2, bf2,
                g2, b2):
    wspecs = [_fullspec((HID, HID)), _fullspec((1, HID)),
              _fullspec((1, HID)), _fullspec((1, HID)),
              _fullspec((HID, FFN)), _fullspec((1, FFN)),
              _fullspec((FFN, HID)), _fullspec((1, HID)),
              _fullspec((1, HID)), _fullspec((1, HID))]
    return pl.pallas_call(
        _node_final_body, grid=(N // NB,),
        in_specs=[_rowspec(NB, HID),
                  pl.BlockSpec((2, NB, HID), lambda i: (0, i, 0)),
                  pl.BlockSpec((2, NB, HID), lambda i: (0, i, 0)),
                  _rowspec(NB, 1), _rowspec(NB, 1)] + wspecs,
        out_specs=[_rowspec(NB, HID), _fullspec((1, HID))],
        out_shape=[jax.ShapeDtypeStruct((N, HID), jnp.int32),
                   jax.ShapeDtypeStruct((1, HID), f32)],
    )(v, aggp, denp, nf0, rad, Wo, bo, g1, b1, Wf1, bf1, Wf2, bf2, g2, b2)


def _sigmoid(x):
    return 1.0 / (1.0 + jnp.exp(-x))


def _readout_body(xs_r, xd_r, d_r, it_r, ef0_r,
                  W21_r, b21_r, W22_r, b22_r, W31_r, b31_r, W32_r, b32_r,
                  y_r):
    x = _unpack64(xs_r[:, :64]) + _unpack64(xd_r[:, :64])
    rsum = (lax.bitcast_convert_type(xs_r[:, 64:65], f32)
            + lax.bitcast_convert_type(xd_r[:, 64:65], f32))
    dist = d_r[...]
    d = dist - rsum
    t = d * 1.25
    V0 = -0.045 * jnp.exp(-(t * t))
    V1 = 0.8 * jnp.where(d < 0, d * d, 0.0)
    V2 = -0.035 * (jnp.where((d > 0) & (d < 2.5), -0.4 * (d - 2.5), 0.0)
                   + jnp.where(d <= 0, 1.0, 0.0))
    V3 = -0.6 * (jnp.where((d > -0.6) & (d < 0), (-5.0 / 3.0) * d, 0.0)
                 + jnp.where(d <= -0.6, 1.0, 0.0))
    mask = jnp.where(dist < CUT, 1.0, 0.0) * ef0_r[...]
    w2 = _sigmoid(_dot(jnp.maximum(_dot(x, W21_r[...]) + b21_r[...], 0.0),
                       W22_r[...]) + b22_r[...]) + 0.5
    w3 = _sigmoid(_dot(jnp.maximum(_dot(x, W31_r[...]) + b31_r[...], 0.0),
                       W32_r[...]) + b32_r[...]) + 0.5
    it1 = it_r[:, 1:2]
    it2 = it_r[:, 2:3]
    t2 = (w2[:, 0:1] * V0 + w2[:, 1:2] * V1 + it1 * w2[:, 2:3] * V2
          + it2 * w2[:, 3:4] * V3)
    t3 = (w3[:, 0:1] * V0 + w3[:, 1:2] * V1 + it1 * w3[:, 2:3] * V2
          + it2 * w3[:, 3:4] * V3)
    p2 = jnp.sum(mask * t2) * 0.5
    p3 = jnp.sum(mask * t3) * 0.5

    @pl.when(pl.program_id(0) == 0)
    def _():
        y_r[...] = jnp.zeros_like(y_r)

    lane = lax.broadcasted_iota(jnp.int32, (1, HID), 1)
    y_r[...] += jnp.where(lane == 0, p2, 0.0) + jnp.where(lane == 1, p3, 0.0)


def _readout(xs, xd, dist, itp, ef0, W21, b21, W22, b22, W31, b31, W32, b32):
    return pl.pallas_call(
        _readout_body, grid=(E // EB,),
        in_specs=[_rowspec(EB, HID), _rowspec(EB, HID), _rowspec(EB, 1),
                  _rowspec(EB, 8), _rowspec(EB, 1),
                  _fullspec((HID, HID)), _fullspec((1, HID)),
                  _fullspec((HID, 8)), _fullspec((1, 8)),
                  _fullspec((HID, HID)), _fullspec((1, HID)),
                  _fullspec((HID, 8)), _fullspec((1, 8))],
        out_specs=_fullspec((1, HID)),
        out_shape=jax.ShapeDtypeStruct((1, HID), f32),
    )(xs, xd, dist, itp, ef0, W21, b21, W22, b22, W31, b31, W32, b32)


def _head_body(z_r, y_r, nrot_r, W11_r, b11_r, W12_r, b12_r,
               W41_r, b41_r, W42_r, b42_r, out_r):
    z = z_r[...]
    y1 = (_dot(jnp.maximum(_dot(z, W11_r[...]) + b11_r[...], 0.0),
               W12_r[...]) + b12_r[...])[0, 0]
    w4 = _sigmoid((_dot(jnp.maximum(_dot(z, W41_r[...]) + b41_r[...], 0.0),
                        W42_r[...]) + b42_r[...])[0, 0]) + 0.5
    y2 = y_r[0, 0]
    y3 = y_r[0, 1] / (1.0 + w4 * 0.05846 * nrot_r[0, 0])
    lane = lax.broadcasted_iota(jnp.int32, (1, HID), 1)
    out_r[...] = (jnp.where(lane == 0, y1, 0.0)
                  + jnp.where(lane == 1, y2, 0.0)
                  + jnp.where(lane == 2, y3, 0.0))


def _head(z, y, nrot, W11, b11, W12, b12, W41, b41, W42, b42):
    return pl.pallas_call(
        _head_body,
        out_shape=jax.ShapeDtypeStruct((1, HID), f32),
    )(z, y, nrot, W11, b11, W12, b12, W41, b41, W42, b42)


def kernel(node_feature, edge_feature, vdw_radii, distance, interaction_type,
           edge_index, n_rot, params):
    p = params
    src = edge_index[0].astype(jnp.int32)
    dst = edge_index[1].astype(jnp.int32)
    nf_pad = jnp.pad(node_feature, ((0, 0), (0, 1)))
    ef_pad = jnp.pad(edge_feature, ((0, 0), (0, 1)))
    Wv = jnp.pad(p['Wv_emb'], ((0, 1), (0, 0)))
    We = jnp.pad(p['We_emb'], ((0, 1), (0, 0)))
    dist = distance[:, None]
    itp = jnp.pad(interaction_type, ((0, 0), (0, 5)))
    ef0 = edge_feature[:, 0:1]
    nf0 = node_feature[:, 0:1]
    rad = vdw_radii[:, None]
    nrot = jnp.asarray(n_rot, f32).reshape(1, 1)

    def row(x):
        return x.reshape(1, -1)

    v, q_t, kv_t = _node_embed(nf_pad, Wv, row(p['bv_emb']),
                               p['Wq'][0], row(p['bq'][0]),
                               p['Wk'][0], row(p['bk'][0]),
                               p['Wvv'][0], row(p['bvv'][0]))
    e, env = _edge_embed(ef_pad, dist, We, row(p['be_emb']))

    vfin = None
    z = None
    for l in range(NLAYERS):
        ksvs, qd = _gather_layer(kv_t, q_t, src, dst)
        score, contrib, arep = _edge_attn(e, ksvs, qd, env,
                                          p['Wee'][l], row(p['bee'][l]))
        aggf, denf = _scatter(contrib, arep, dst)
        e = _edge_up(e, score, p['Woe'][l], row(p['boe'][l]),
                     row(p['g1e'][l]), row(p['b1e'][l]),
                     p['Wf1e'][l], row(p['bf1e'][l]),
                     p['Wf2e'][l], row(p['bf2e'][l]),
                     row(p['g2e'][l]), row(p['b2e'][l]))
        aggp = aggf.reshape(2, NPAD, HID)
        denp = denf.reshape(2, NPAD, HID)
        nw = (p['Wo'][l], row(p['bo'][l]), row(p['g1v'][l]), row(p['b1v'][l]),
              p['Wf1v'][l], row(p['bf1v'][l]), p['Wf2v'][l], row(p['bf2v'][l]),
              row(p['g2v'][l]), row(p['b2v'][l]))
        if l < NLAYERS - 1:
            v, q_t, kv_t = _node_layer(v, aggp, denp, *nw,
                                       p['Wq'][l + 1], row(p['bq'][l + 1]),
                                       p['Wk'][l + 1], row(p['bk'][l + 1]),
                                       p['Wvv'][l + 1], row(p['bvv'][l + 1]))
        else:
            vfin, z = _node_final(v, aggp, denp, nf0, rad, *nw)

    xs, xd = _gather_fin(vfin, vfin, src, dst)
    y23 = _readout(xs, xd, dist, itp, ef0,
                   p['r2_W1'], row(p['r2_b1']),
                   jnp.pad(p['r2_W2'], ((0, 0), (0, 4))),
                   row(jnp.pad(p['r2_b2'], (0, 4))),
                   p['r3_W1'], row(p['r3_b1']),
                   jnp.pad(p['r3_W2'], ((0, 0), (0, 4))),
                   row(jnp.pad(p['r3_b2'], (0, 4))))
    out = _head(z, y23, nrot,
                p['r1_W1'], row(p['r1_b1']),
                jnp.pad(p['r1_W2'], ((0, 0), (0, 7))),
                row(jnp.pad(p['r1_b2'], (0, 7))),
                p['r4_W1'], row(p['r4_b1']),
                jnp.pad(p['r4_W2'], ((0, 0), (0, 7))),
                row(jnp.pad(p['r4_b2'], (0, 7))))
    return out[0, :3]
